# baseline (speedup 1.0000x reference)
import functools

import jax
import jax.numpy as jnp
from jax import lax
from jax.experimental import pallas as pl
from jax.experimental.pallas import tpu as pltpu
from jax.experimental.pallas import tpu_sc as plsc

_N = 50000
_E = 800000
_H = 64

_NC = 2
_NS = 16
_NW = _NC * _NS

_CHUNK = 128
_GROUP = 5
_GE = _CHUNK * _GROUP
_NR = _E // _CHUNK
_NG = _E // _GE

_HF = _H // _NC
_TROWS = _N // _NS
_CROWS = 3200

_f32 = jnp.float32
_bf16 = jnp.bfloat16
_i32 = jnp.int32


def _mesh():
    return plsc.VectorSubcoreMesh(core_axis_name="c", subcore_axis_name="s")


def _sc_gather(h, row2d, col2d):
    ngc = row2d.shape[0]
    ec = ngc * _GE

    @functools.partial(
        pl.kernel,
        out_type=(
            jax.ShapeDtypeStruct((ec, _H), _bf16),
            jax.ShapeDtypeStruct((ec, _H), _bf16),
        ),
        mesh=_mesh(),
        compiler_params=pltpu.CompilerParams(use_tc_tiling_on_sc=False, needs_layout_passes=False),
        scratch_types=[
            pltpu.VMEM((_GROUP, _CHUNK), _i32),
            pltpu.VMEM((_GROUP, _CHUNK), _i32),
            pltpu.VMEM((_GE, _H), _bf16),
            pltpu.VMEM((_GE, _H), _bf16),
            pltpu.SemaphoreType.DMA,
            pltpu.SemaphoreType.DMA,
            pltpu.SemaphoreType.DMA,
        ],
    )
    def k(h_hbm, row_hbm, col_hbm, outr, outc, idxA, idxB, bufA, bufB,
          semA, semB, semS):
        c = lax.axis_index("c")
        s = lax.axis_index("s")
        base, rem = divmod(ngc, _NS)
        npairs = base // 2
        ng = base + jnp.where(s < rem, 1, 0)

        def stream(arr_hbm, out_hbm):
            def pair(jj, carry):
                ga = s + (2 * jj) * _NS
                gb = s + (2 * jj + 1) * _NS
                pltpu.sync_copy(arr_hbm.at[ga], idxA)
                dA = [pltpu.async_copy(
                    h_hbm.at[idxA.at[t]], bufA.at[pl.ds(t * _CHUNK, _CHUNK)],
                    semA) for t in range(_GROUP)]
                pltpu.sync_copy(arr_hbm.at[gb], idxB)
                dB = [pltpu.async_copy(
                    h_hbm.at[idxB.at[t]], bufB.at[pl.ds(t * _CHUNK, _CHUNK)],
                    semB) for t in range(_GROUP)]
                for d in dA:
                    d.wait()
                dS = pltpu.async_copy(bufA, out_hbm.at[pl.ds(ga * _GE, _GE)], semS)
                for d in dB:
                    d.wait()
                pltpu.sync_copy(bufB, out_hbm.at[pl.ds(gb * _GE, _GE)])
                dS.wait()
                return carry

            lax.fori_loop(0, npairs, pair, 0)

            def tail(j, carry):
                g = s + j * _NS
                pltpu.sync_copy(arr_hbm.at[g], idxA)
                ds = [pltpu.async_copy(
                    h_hbm.at[idxA.at[t]], bufA.at[pl.ds(t * _CHUNK, _CHUNK)],
                    semA) for t in range(_GROUP)]
                for d in ds:
                    d.wait()
                pltpu.sync_copy(bufA, out_hbm.at[pl.ds(g * _GE, _GE)])
                return carry

            lax.fori_loop(2 * npairs, ng, tail, 0)

        @pl.when(c == 0)
        def _():
            stream(row_hbm, outr)

        @pl.when(c == 1)
        def _():
            stream(col_hbm, outc)

    return k(h, row2d, col2d)


def _sc_scatter(e, col2d):
    QF = 16
    ngc = col2d.shape[0]

    @functools.partial(
        pl.kernel,
        out_type=jax.ShapeDtypeStruct((_N, _H), _f32),
        mesh=_mesh(),
        compiler_params=pltpu.CompilerParams(use_tc_tiling_on_sc=False, needs_layout_passes=False),
        scratch_types=[
            pltpu.VMEM((_GROUP, _CHUNK), _i32),
            pltpu.VMEM((_GROUP, _CHUNK), _i32),
            pltpu.VMEM((_GE, QF), _f32),
            pltpu.VMEM((_GE, QF), _f32),
            pltpu.VMEM((1000, QF), _f32),
            pltpu.VMEM((1000, QF), _f32),
            pltpu.VMEM_SHARED((_N, QF), _f32),
            pltpu.SemaphoreType.DMA,
            pltpu.SemaphoreType.DMA,
            pltpu.SemaphoreType.DMA,
        ],
    )
    def k(e_hbm, col_hbm, out_hbm, idxA, idxB, ebA, ebB, bounce, zbuf, acc,
          semA, semB, semU):
        c = lax.axis_index("c")
        s = lax.axis_index("s")
        z = jnp.zeros((16,), _f32)

        def zb(i, carry):
            zbuf[i, :] = z
            return carry

        lax.fori_loop(0, 1000, zb, 0)

        nz = (_N // 1000 - s + _NS - 1) // _NS

        def zc(j, carry):
            pltpu.sync_copy(zbuf, acc.at[pl.ds((s + j * _NS) * 1000, 1000)])
            return carry

        lax.fori_loop(0, nz, zc, 0)

        base, rem = divmod(ngc, _NS)
        npairs = base // 2
        ng = base + jnp.where(s < rem, 1, 0)

        for p in range(2):
            f0 = (c + 2 * p) * QF
            plsc.subcore_barrier()

            def prefA(g):
                pltpu.async_copy(col_hbm.at[g], idxA, semA)
                pltpu.async_copy(
                    e_hbm.at[pl.ds(g * _GE, _GE), pl.ds(f0, QF)], ebA, semA)

            def waitA(g):
                pltpu.make_async_copy(col_hbm.at[g], idxA, semA).wait()
                pltpu.make_async_copy(
                    e_hbm.at[pl.ds(g * _GE, _GE), pl.ds(f0, QF)], ebA,
                    semA).wait()

            prefA(s)

            def pair(jj, carry):
                ga = s + (2 * jj) * _NS
                gb = ga + _NS
                waitA(ga)
                pltpu.async_copy(col_hbm.at[gb], idxB, semB)
                pltpu.async_copy(
                    e_hbm.at[pl.ds(gb * _GE, _GE), pl.ds(f0, QF)], ebB, semB)
                dU = [pltpu.async_copy(
                    ebA.at[pl.ds(t * _CHUNK, _CHUNK)], acc.at[idxA.at[t]],
                    semU, add=True) for t in range(_GROUP)]
                for d in dU:
                    d.wait()

                @pl.when(jj < npairs - 1)
                def _():
                    prefA(ga + 2 * _NS)

                pltpu.make_async_copy(col_hbm.at[gb], idxB, semB).wait()
                pltpu.make_async_copy(
                    e_hbm.at[pl.ds(gb * _GE, _GE), pl.ds(f0, QF)], ebB,
                    semB).wait()
                dV = [pltpu.async_copy(
                    ebB.at[pl.ds(t * _CHUNK, _CHUNK)], acc.at[idxB.at[t]],
                    semU, add=True) for t in range(_GROUP)]
                for d in dV:
                    d.wait()
                return carry

            lax.fori_loop(0, npairs, pair, 0)

            def tailS(j, carry):
                ga = s + j * _NS
                pltpu.sync_copy(col_hbm.at[ga], idxA)
                pltpu.sync_copy(
                    e_hbm.at[pl.ds(ga * _GE, _GE), pl.ds(f0, QF)], ebA)
                ds = [pltpu.async_copy(
                    ebA.at[pl.ds(t * _CHUNK, _CHUNK)], acc.at[idxA.at[t]],
                    semU, add=True) for t in range(_GROUP)]
                for d in ds:
                    d.wait()
                return carry

            lax.fori_loop(2 * npairs, ng, tailS, 0)

            plsc.subcore_barrier()

            def co(j, carry):
                r0 = (s + j * _NS) * 1000
                pltpu.sync_copy(acc.at[pl.ds(r0, 1000)], bounce)
                if p == 0:
                    pltpu.sync_copy(zbuf, acc.at[pl.ds(r0, 1000)])
                pltpu.sync_copy(bounce, out_hbm.at[pl.ds(r0, 1000), pl.ds(f0, QF)])
                return carry

            lax.fori_loop(0, nz, co, 0)

    return k(e, col2d)


def _sc_count(col2d):
    @functools.partial(
        pl.kernel,
        out_type=jax.ShapeDtypeStruct((_NC, _CROWS, 16), _f32),
        mesh=_mesh(),
        compiler_params=pltpu.CompilerParams(use_tc_tiling_on_sc=False, needs_layout_passes=False),
        scratch_types=[
            pltpu.VMEM((_CROWS, 16), _f32),
            pltpu.VMEM((_GROUP, _CHUNK), _i32),
            pltpu.VMEM((25, 128), _i32),
            pltpu.VMEM((200, 16), _f32),
            pltpu.VMEM_SHARED((_CROWS, 16), _f32),
        ],
    )
    def k(col_hbm, out_hbm, local, idxs, iotaref, bounce, acc):
        c = lax.axis_index("c")
        s = lax.axis_index("s")
        w = s * _NC + c
        z = jnp.zeros((16,), _f32)
        ones = jnp.ones((16,), _f32)
        ar = jnp.arange(16, dtype=_i32)

        def z1(i, carry):
            local[i, :] = z
            return carry

        lax.fori_loop(0, _CROWS, z1, 0)

        def z2(i, carry):
            bounce[i, :] = z
            return carry

        lax.fori_loop(0, 200, z2, 0)
        pltpu.sync_copy(bounce, acc.at[pl.ds(s * 200, 200)])

        def bi(j, carry):
            for t in range(8):
                iotaref[j, pl.ds(t * 16, 16)] = ar + (j * 128 + t * 16)
            return carry

        lax.fori_loop(0, 25, bi, 0)
        plsc.subcore_barrier()

        base, rem = divmod(_NG, _NW)
        ngr = base + jnp.where(w < rem, 1, 0)

        def body(j, carry):
            g = w + j * _NW
            pltpu.sync_copy(col_hbm.at[g], idxs)
            for t in range(_GROUP):
                for q in range(_CHUNK // 16):
                    iv = idxs[t, pl.ds(q * 16, 16)]
                    rr = lax.shift_right_logical(iv, 4)
                    cc = lax.bitwise_and(iv, 15)
                    plsc.addupdate_scatter(local, [rr, cc], ones)
            return carry

        lax.fori_loop(0, ngr, body, 0)

        def mg(j, carry):
            pltpu.sync_copy(
                local.at[pl.ds(j * 128, 128)], acc.at[iotaref.at[j]], add=True)
            return carry

        lax.fori_loop(0, 25, mg, 0)
        plsc.subcore_barrier()

        pltpu.sync_copy(acc.at[pl.ds(s * 200, 200)], bounce)
        pltpu.sync_copy(bounce, out_hbm.at[c, pl.ds(s * 200, 200)])

    return k(col2d)


def _gelu(x):
    return x * 0.5 * (1.0 + lax.erf(x * 0.7071067811865476))


def _lnorm(x, g, b):
    n = x.shape[-1]
    sx = jnp.sum(x, axis=-1, keepdims=True)
    sxx = jnp.sum(x * x, axis=-1, keepdims=True)
    m = sx * (1.0 / n)
    v = sxx * (1.0 / n) - m * m
    r = lax.rsqrt(v + 1e-5)
    return (x - m) * (r * g) + b


def _full(a):
    return pl.BlockSpec(a.shape, lambda i: (0,) * a.ndim)


def _tc_encode(x8, W8, b, g, bt):
    R = 1000

    def body(x_r, W_r, b_r, g_r, bt_r, out_r, outb_r):
        t = jnp.dot(x_r[...], W_r[...], preferred_element_type=_f32) + b_r[...]
        t = _gelu(_lnorm(t, g_r[...], bt_r[...]))
        out_r[...] = t
        outb_r[...] = t.astype(_bf16)

    blk = pl.BlockSpec((R, _H), lambda i: (i, 0))
    return pl.pallas_call(
        body,
        grid=(_N // R,),
        in_specs=[pl.BlockSpec((R, 8), lambda i: (i, 0)),
                  _full(W8), _full(b), _full(g), _full(bt)],
        out_specs=(blk, blk),
        out_shape=(jax.ShapeDtypeStruct((_N, _H), _f32),
                   jax.ShapeDtypeStruct((_N, _H), _bf16)),
    )(x8, W8, b, g, bt)


def _tc_edge_encode(ea, W, b):
    R = 1600

    def body(a_r, W_r, b_r, out_r):
        out_r[...] = jnp.dot(a_r[...], W_r[...], preferred_element_type=_f32) + b_r[...]

    return pl.pallas_call(
        body,
        grid=(_E // R,),
        in_specs=[pl.BlockSpec((R, 8), lambda i: (i, 0)), _full(W), _full(b)],
        out_specs=pl.BlockSpec((R, _H), lambda i: (i, 0)),
        out_shape=jax.ShapeDtypeStruct((_E, _H), _f32),
    )(ea, W, b)


def _tc_edge_mlp(ghr, ghc, e, W1hb, W1e, b1, g1, bt1, W2, b2, g2, bt2):
    R = 1600

    def body(ghr_r, ghc_r, e_r, W1hb_r, W1e_r, b1_r, g1_r, bt1_r, W2_r, b2_r,
             g2_r, bt2_r, out_r):
        gin = jnp.concatenate([ghr_r[...], ghc_r[...]], axis=1)
        t = (jnp.dot(gin, W1hb_r[...], preferred_element_type=_f32)
             + jnp.dot(e_r[...], W1e_r[...], preferred_element_type=_f32)
             + b1_r[...])
        t = _gelu(_lnorm(t, g1_r[...], bt1_r[...]))
        t = jnp.dot(t, W2_r[...], preferred_element_type=_f32) + b2_r[...]
        t = _lnorm(t, g2_r[...], bt2_r[...])
        out_r[...] = e_r[...] + t

    ec = e.shape[0]
    blk = pl.BlockSpec((R, _H), lambda i: (i, 0))
    return pl.pallas_call(
        body,
        grid=(ec // R,),
        in_specs=[blk, blk, blk, _full(W1hb), _full(W1e), _full(b1),
                  _full(g1), _full(bt1),
                  _full(W2), _full(b2), _full(g2), _full(bt2)],
        out_specs=blk,
        out_shape=jax.ShapeDtypeStruct((ec, _H), _f32),
    )(ghr, ghc, e, W1hb, W1e, b1, g1, bt1, W2, b2, g2, bt2)


def _tc_node_mlp(h, aggA, aggB, cnt, W1, b1, g1, bt1, W2, b2, g2, bt2):
    R = 1000

    def body(h_r, aggA_r, aggB_r, cnt_r, W1_r, b1_r, g1_r, bt1_r, W2_r, b2_r,
             g2_r, bt2_r, out_r, outb_r):
        csum = cnt_r[0] + cnt_r[1]
        inv = 1.0 / jnp.maximum(csum, 1.0)
        agg = aggA_r[...] + aggB_r[...]
        nin = jnp.concatenate([h_r[...], agg * inv], axis=1)
        u = jnp.dot(nin, W1_r[...], preferred_element_type=_f32) + b1_r[...]
        u = _gelu(_lnorm(u, g1_r[...], bt1_r[...]))
        u = jnp.dot(u, W2_r[...], preferred_element_type=_f32) + b2_r[...]
        u = _lnorm(u, g2_r[...], bt2_r[...])
        hn = h_r[...] + u
        out_r[...] = hn
        outb_r[...] = hn.astype(_bf16)

    blk = pl.BlockSpec((R, _H), lambda i: (i, 0))
    return pl.pallas_call(
        body,
        grid=(_N // R,),
        in_specs=[blk, blk, blk, pl.BlockSpec((2, R, 1), lambda i: (0, i, 0)),
                  _full(W1), _full(b1), _full(g1), _full(bt1),
                  _full(W2), _full(b2), _full(g2), _full(bt2)],
        out_specs=(blk, blk),
        out_shape=(jax.ShapeDtypeStruct((_N, _H), _f32),
                   jax.ShapeDtypeStruct((_N, _H), _bf16)),
    )(h, aggA, aggB, cnt, W1, b1, g1, bt1, W2, b2, g2, bt2)


def _tc_decode(h, W1, b1, W2, b2):
    R = 1000

    def body(h_r, W1_r, b1_r, W2_r, b2_r, out_r):
        t = _gelu(jnp.dot(h_r[...], W1_r[...], preferred_element_type=_f32)
                  + b1_r[...])
        out_r[...] = jnp.dot(t, W2_r[...], preferred_element_type=_f32) + b2_r[...]

    return pl.pallas_call(
        body,
        grid=(_N // R,),
        in_specs=[pl.BlockSpec((R, _H), lambda i: (i, 0)),
                  _full(W1), _full(b1), _full(W2), _full(b2)],
        out_specs=pl.BlockSpec((R, 4), lambda i: (i, 0)),
        out_shape=jax.ShapeDtypeStruct((_N, 4), _f32),
    )(h, W1, b1, W2, b2)


def kernel(x, edge_index, edge_attr, enc_W, enc_b, enc_g, enc_bt, ee_W, ee_b,
           eW1, eb1, eg1, ebt1, eW2, eb2, eg2, ebt2,
           nW1, nb1, ng1, nbt1, nW2, nb2, ng2, nbt2,
           dW1, db1, dW2, db2):
    row3 = edge_index[0].reshape(_NG, _GROUP, _CHUNK)
    col3 = edge_index[1].reshape(_NG, _GROUP, _CHUNK)
    ngh = _NG // 2
    eh = ngh * _GE
    rowA, rowB = row3[:ngh], row3[ngh:]
    colA, colB = col3[:ngh], col3[ngh:]
    x8 = jnp.pad(x, ((0, 0), (0, 1)))
    W8 = jnp.pad(enc_W, ((0, 1), (0, 0)))

    cnt_raw = _sc_count(col3)
    cnt = cnt_raw.reshape(_NC, _CROWS * 16)[:, :_N].reshape(_NC, _N, 1)

    h, hb = _tc_encode(x8, W8, enc_b, enc_g, enc_bt)
    eW1hb = eW1[:, :2 * _H, :].astype(_bf16)
    eW1e = eW1[:, 2 * _H:, :]
    e = _tc_edge_encode(edge_attr, ee_W, ee_b)
    eA, eB = e[:eh], e[eh:]

    L = eW1.shape[0]
    for i in range(L):
        ghrA, ghcA = _sc_gather(hb, rowA, colA)
        ghrB, ghcB = _sc_gather(hb, rowB, colB)
        eA = _tc_edge_mlp(ghrA, ghcA, eA, eW1hb[i], eW1e[i], eb1[i], eg1[i],
                          ebt1[i], eW2[i], eb2[i], eg2[i], ebt2[i])
        aggA = _sc_scatter(eA, colA)
        eB = _tc_edge_mlp(ghrB, ghcB, eB, eW1hb[i], eW1e[i], eb1[i], eg1[i],
                          ebt1[i], eW2[i], eb2[i], eg2[i], ebt2[i])
        aggB = _sc_scatter(eB, colB)
        h, hb = _tc_node_mlp(h, aggA, aggB, cnt, nW1[i], nb1[i], ng1[i],
                             nbt1[i], nW2[i], nb2[i], ng2[i], nbt2[i])

    return _tc_decode(h, dW1, db1, dW2, db2)


# speedup vs baseline: 1.0875x; 1.0875x over previous
import functools

import jax
import jax.numpy as jnp
from jax import lax
from jax.experimental import pallas as pl
from jax.experimental.pallas import tpu as pltpu
from jax.experimental.pallas import tpu_sc as plsc

_N = 50000
_E = 800000
_H = 64

_NC = 2
_NS = 16
_NW = _NC * _NS

_CHUNK = 128
_GROUP = 5
_GE = _CHUNK * _GROUP
_NR = _E // _CHUNK
_NG = _E // _GE

_HF = _H // _NC
_TROWS = _N // _NS
_CROWS = 3200

_f32 = jnp.float32
_bf16 = jnp.bfloat16
_i32 = jnp.int32


def _mesh():
    return plsc.VectorSubcoreMesh(core_axis_name="c", subcore_axis_name="s")


def _sc_gather(h, row2d, col2d):
    ngc = row2d.shape[0]
    ec = ngc * _GE

    @functools.partial(
        pl.kernel,
        out_type=(
            jax.ShapeDtypeStruct((ec, _H), _f32),
            jax.ShapeDtypeStruct((ec, _H), _f32),
        ),
        mesh=_mesh(),
        compiler_params=pltpu.CompilerParams(use_tc_tiling_on_sc=False, needs_layout_passes=False),
        scratch_types=[
            pltpu.VMEM((_GROUP, _CHUNK), _i32),
            pltpu.VMEM((_GROUP, _CHUNK), _i32),
            pltpu.VMEM((_GE, _H), _f32),
            pltpu.VMEM((_GE, _H), _f32),
            pltpu.SemaphoreType.DMA,
            pltpu.SemaphoreType.DMA,
            pltpu.SemaphoreType.DMA,
        ],
    )
    def k(h_hbm, row_hbm, col_hbm, outr, outc, idxA, idxB, bufA, bufB,
          semA, semB, semS):
        c = lax.axis_index("c")
        s = lax.axis_index("s")
        base, rem = divmod(ngc, _NS)
        npairs = base // 2
        ng = base + jnp.where(s < rem, 1, 0)

        def stream(arr_hbm, out_hbm):
            def pair(jj, carry):
                ga = s + (2 * jj) * _NS
                gb = s + (2 * jj + 1) * _NS
                pltpu.sync_copy(arr_hbm.at[ga], idxA)
                dA = [pltpu.async_copy(
                    h_hbm.at[idxA.at[t]], bufA.at[pl.ds(t * _CHUNK, _CHUNK)],
                    semA) for t in range(_GROUP)]
                pltpu.sync_copy(arr_hbm.at[gb], idxB)
                dB = [pltpu.async_copy(
                    h_hbm.at[idxB.at[t]], bufB.at[pl.ds(t * _CHUNK, _CHUNK)],
                    semB) for t in range(_GROUP)]
                for d in dA:
                    d.wait()
                dS = pltpu.async_copy(bufA, out_hbm.at[pl.ds(ga * _GE, _GE)], semS)
                for d in dB:
                    d.wait()
                pltpu.sync_copy(bufB, out_hbm.at[pl.ds(gb * _GE, _GE)])
                dS.wait()
                return carry

            lax.fori_loop(0, npairs, pair, 0)

            def tail(j, carry):
                g = s + j * _NS
                pltpu.sync_copy(arr_hbm.at[g], idxA)
                ds = [pltpu.async_copy(
                    h_hbm.at[idxA.at[t]], bufA.at[pl.ds(t * _CHUNK, _CHUNK)],
                    semA) for t in range(_GROUP)]
                for d in ds:
                    d.wait()
                pltpu.sync_copy(bufA, out_hbm.at[pl.ds(g * _GE, _GE)])
                return carry

            lax.fori_loop(2 * npairs, ng, tail, 0)

        @pl.when(c == 0)
        def _():
            stream(row_hbm, outr)

        @pl.when(c == 1)
        def _():
            stream(col_hbm, outc)

    return k(h, row2d, col2d)


def _sc_scatter(e, col2d):
    QF = 16
    ngc = col2d.shape[0]

    @functools.partial(
        pl.kernel,
        out_type=jax.ShapeDtypeStruct((_N, _H), _f32),
        mesh=_mesh(),
        compiler_params=pltpu.CompilerParams(use_tc_tiling_on_sc=False, needs_layout_passes=False),
        scratch_types=[
            pltpu.VMEM((_GROUP, _CHUNK), _i32),
            pltpu.VMEM((_GROUP, _CHUNK), _i32),
            pltpu.VMEM((_GE, QF), _f32),
            pltpu.VMEM((_GE, QF), _f32),
            pltpu.VMEM((1000, QF), _f32),
            pltpu.VMEM((1000, QF), _f32),
            pltpu.VMEM_SHARED((_N, QF), _f32),
            pltpu.SemaphoreType.DMA,
            pltpu.SemaphoreType.DMA,
            pltpu.SemaphoreType.DMA,
        ],
    )
    def k(e_hbm, col_hbm, out_hbm, idxA, idxB, ebA, ebB, bounce, zbuf, acc,
          semA, semB, semU):
        c = lax.axis_index("c")
        s = lax.axis_index("s")
        z = jnp.zeros((16,), _f32)

        def zb(i, carry):
            zbuf[i, :] = z
            return carry

        lax.fori_loop(0, 1000, zb, 0)

        nz = (_N // 1000 - s + _NS - 1) // _NS

        def zc(j, carry):
            pltpu.sync_copy(zbuf, acc.at[pl.ds((s + j * _NS) * 1000, 1000)])
            return carry

        lax.fori_loop(0, nz, zc, 0)

        base, rem = divmod(ngc, _NS)
        npairs = base // 2
        ng = base + jnp.where(s < rem, 1, 0)

        for p in range(2):
            f0 = (c + 2 * p) * QF
            plsc.subcore_barrier()

            def prefA(g):
                pltpu.async_copy(col_hbm.at[g], idxA, semA)
                pltpu.async_copy(
                    e_hbm.at[pl.ds(g * _GE, _GE), pl.ds(f0, QF)], ebA, semA)

            def waitA(g):
                pltpu.make_async_copy(col_hbm.at[g], idxA, semA).wait()
                pltpu.make_async_copy(
                    e_hbm.at[pl.ds(g * _GE, _GE), pl.ds(f0, QF)], ebA,
                    semA).wait()

            prefA(s)

            def pair(jj, carry):
                ga = s + (2 * jj) * _NS
                gb = ga + _NS
                waitA(ga)
                pltpu.async_copy(col_hbm.at[gb], idxB, semB)
                pltpu.async_copy(
                    e_hbm.at[pl.ds(gb * _GE, _GE), pl.ds(f0, QF)], ebB, semB)
                dU = [pltpu.async_copy(
                    ebA.at[pl.ds(t * _CHUNK, _CHUNK)], acc.at[idxA.at[t]],
                    semU, add=True) for t in range(_GROUP)]
                for d in dU:
                    d.wait()

                @pl.when(jj < npairs - 1)
                def _():
                    prefA(ga + 2 * _NS)

                pltpu.make_async_copy(col_hbm.at[gb], idxB, semB).wait()
                pltpu.make_async_copy(
                    e_hbm.at[pl.ds(gb * _GE, _GE), pl.ds(f0, QF)], ebB,
                    semB).wait()
                dV = [pltpu.async_copy(
                    ebB.at[pl.ds(t * _CHUNK, _CHUNK)], acc.at[idxB.at[t]],
                    semU, add=True) for t in range(_GROUP)]
                for d in dV:
                    d.wait()
                return carry

            lax.fori_loop(0, npairs, pair, 0)

            def tailS(j, carry):
                ga = s + j * _NS
                pltpu.sync_copy(col_hbm.at[ga], idxA)
                pltpu.sync_copy(
                    e_hbm.at[pl.ds(ga * _GE, _GE), pl.ds(f0, QF)], ebA)
                ds = [pltpu.async_copy(
                    ebA.at[pl.ds(t * _CHUNK, _CHUNK)], acc.at[idxA.at[t]],
                    semU, add=True) for t in range(_GROUP)]
                for d in ds:
                    d.wait()
                return carry

            lax.fori_loop(2 * npairs, ng, tailS, 0)

            plsc.subcore_barrier()

            def co(j, carry):
                r0 = (s + j * _NS) * 1000
                pltpu.sync_copy(acc.at[pl.ds(r0, 1000)], bounce)
                if p == 0:
                    pltpu.sync_copy(zbuf, acc.at[pl.ds(r0, 1000)])
                pltpu.sync_copy(bounce, out_hbm.at[pl.ds(r0, 1000), pl.ds(f0, QF)])
                return carry

            lax.fori_loop(0, nz, co, 0)

    return k(e, col2d)


def _sc_count(col2d):
    @functools.partial(
        pl.kernel,
        out_type=jax.ShapeDtypeStruct((_NC, _CROWS, 16), _f32),
        mesh=_mesh(),
        compiler_params=pltpu.CompilerParams(use_tc_tiling_on_sc=False, needs_layout_passes=False),
        scratch_types=[
            pltpu.VMEM((_CROWS, 16), _f32),
            pltpu.VMEM((_GROUP, _CHUNK), _i32),
            pltpu.VMEM((25, 128), _i32),
            pltpu.VMEM((200, 16), _f32),
            pltpu.VMEM_SHARED((_CROWS, 16), _f32),
        ],
    )
    def k(col_hbm, out_hbm, local, idxs, iotaref, bounce, acc):
        c = lax.axis_index("c")
        s = lax.axis_index("s")
        w = s * _NC + c
        z = jnp.zeros((16,), _f32)
        ones = jnp.ones((16,), _f32)
        ar = jnp.arange(16, dtype=_i32)

        def z1(i, carry):
            local[i, :] = z
            return carry

        lax.fori_loop(0, _CROWS, z1, 0)

        def z2(i, carry):
            bounce[i, :] = z
            return carry

        lax.fori_loop(0, 200, z2, 0)
        pltpu.sync_copy(bounce, acc.at[pl.ds(s * 200, 200)])

        def bi(j, carry):
            for t in range(8):
                iotaref[j, pl.ds(t * 16, 16)] = ar + (j * 128 + t * 16)
            return carry

        lax.fori_loop(0, 25, bi, 0)
        plsc.subcore_barrier()

        base, rem = divmod(_NG, _NW)
        ngr = base + jnp.where(w < rem, 1, 0)

        def body(j, carry):
            g = w + j * _NW
            pltpu.sync_copy(col_hbm.at[g], idxs)
            for t in range(_GROUP):
                for q in range(_CHUNK // 16):
                    iv = idxs[t, pl.ds(q * 16, 16)]
                    rr = lax.shift_right_logical(iv, 4)
                    cc = lax.bitwise_and(iv, 15)
                    plsc.addupdate_scatter(local, [rr, cc], ones)
            return carry

        lax.fori_loop(0, ngr, body, 0)

        def mg(j, carry):
            pltpu.sync_copy(
                local.at[pl.ds(j * 128, 128)], acc.at[iotaref.at[j]], add=True)
            return carry

        lax.fori_loop(0, 25, mg, 0)
        plsc.subcore_barrier()

        pltpu.sync_copy(acc.at[pl.ds(s * 200, 200)], bounce)
        pltpu.sync_copy(bounce, out_hbm.at[c, pl.ds(s * 200, 200)])

    return k(col2d)


def _gelu(x):
    return x * 0.5 * (1.0 + lax.erf(x * 0.7071067811865476))


def _lnorm(x, g, b):
    n = x.shape[-1]
    sx = jnp.sum(x, axis=-1, keepdims=True)
    sxx = jnp.sum(x * x, axis=-1, keepdims=True)
    m = sx * (1.0 / n)
    v = sxx * (1.0 / n) - m * m
    r = lax.rsqrt(v + 1e-5)
    return (x - m) * (r * g) + b


def _full(a):
    return pl.BlockSpec(a.shape, lambda i: (0,) * a.ndim)


def _tc_encode(x8, W8, b, g, bt):
    R = 1000

    def body(x_r, W_r, b_r, g_r, bt_r, out_r):
        t = jnp.dot(x_r[...], W_r[...], preferred_element_type=_f32) + b_r[...]
        out_r[...] = _gelu(_lnorm(t, g_r[...], bt_r[...]))

    return pl.pallas_call(
        body,
        grid=(_N // R,),
        in_specs=[pl.BlockSpec((R, 8), lambda i: (i, 0)),
                  _full(W8), _full(b), _full(g), _full(bt)],
        out_specs=pl.BlockSpec((R, _H), lambda i: (i, 0)),
        out_shape=jax.ShapeDtypeStruct((_N, _H), _f32),
    )(x8, W8, b, g, bt)


def _tc_edge_encode(ea, W, b):
    R = 1600

    def body(a_r, W_r, b_r, out_r):
        out_r[...] = jnp.dot(a_r[...], W_r[...], preferred_element_type=_f32) + b_r[...]

    return pl.pallas_call(
        body,
        grid=(_E // R,),
        in_specs=[pl.BlockSpec((R, 8), lambda i: (i, 0)), _full(W), _full(b)],
        out_specs=pl.BlockSpec((R, _H), lambda i: (i, 0)),
        out_shape=jax.ShapeDtypeStruct((_E, _H), _f32),
    )(ea, W, b)


def _tc_edge_mlp(ghr, ghc, e, W1hb, W1e, b1, g1, bt1, W2, b2, g2, bt2):
    R = 1600

    def body(ghr_r, ghc_r, e_r, W1hb_r, W1e_r, b1_r, g1_r, bt1_r, W2_r, b2_r,
             g2_r, bt2_r, out_r):
        gin = jnp.concatenate([ghr_r[...], ghc_r[...]], axis=1)
        t = (jnp.dot(gin, W1hb_r[...], preferred_element_type=_f32)
             + jnp.dot(e_r[...], W1e_r[...], preferred_element_type=_f32)
             + b1_r[...])
        t = _gelu(_lnorm(t, g1_r[...], bt1_r[...]))
        t = jnp.dot(t, W2_r[...], preferred_element_type=_f32) + b2_r[...]
        t = _lnorm(t, g2_r[...], bt2_r[...])
        out_r[...] = e_r[...] + t

    ec = e.shape[0]
    blk = pl.BlockSpec((R, _H), lambda i: (i, 0))
    return pl.pallas_call(
        body,
        grid=(ec // R,),
        in_specs=[blk, blk, blk, _full(W1hb), _full(W1e), _full(b1),
                  _full(g1), _full(bt1),
                  _full(W2), _full(b2), _full(g2), _full(bt2)],
        out_specs=blk,
        out_shape=jax.ShapeDtypeStruct((ec, _H), _f32),
    )(ghr, ghc, e, W1hb, W1e, b1, g1, bt1, W2, b2, g2, bt2)


def _tc_node_mlp(h, agg, cnt, W1, b1, g1, bt1, W2, b2, g2, bt2):
    R = 1000

    def body(h_r, agg_r, cnt_r, W1_r, b1_r, g1_r, bt1_r, W2_r, b2_r,
             g2_r, bt2_r, out_r):
        csum = cnt_r[0] + cnt_r[1]
        inv = 1.0 / jnp.maximum(csum, 1.0)
        nin = jnp.concatenate([h_r[...], agg_r[...] * inv], axis=1)
        u = jnp.dot(nin, W1_r[...], preferred_element_type=_f32) + b1_r[...]
        u = _gelu(_lnorm(u, g1_r[...], bt1_r[...]))
        u = jnp.dot(u, W2_r[...], preferred_element_type=_f32) + b2_r[...]
        u = _lnorm(u, g2_r[...], bt2_r[...])
        out_r[...] = h_r[...] + u

    blk = pl.BlockSpec((R, _H), lambda i: (i, 0))
    return pl.pallas_call(
        body,
        grid=(_N // R,),
        in_specs=[blk, blk, pl.BlockSpec((2, R, 1), lambda i: (0, i, 0)),
                  _full(W1), _full(b1), _full(g1), _full(bt1),
                  _full(W2), _full(b2), _full(g2), _full(bt2)],
        out_specs=blk,
        out_shape=jax.ShapeDtypeStruct((_N, _H), _f32),
    )(h, agg, cnt, W1, b1, g1, bt1, W2, b2, g2, bt2)


def _tc_decode(h, W1, b1, W2, b2):
    R = 1000

    def body(h_r, W1_r, b1_r, W2_r, b2_r, out_r):
        t = _gelu(jnp.dot(h_r[...], W1_r[...], preferred_element_type=_f32)
                  + b1_r[...])
        out_r[...] = jnp.dot(t, W2_r[...], preferred_element_type=_f32) + b2_r[...]

    return pl.pallas_call(
        body,
        grid=(_N // R,),
        in_specs=[pl.BlockSpec((R, _H), lambda i: (i, 0)),
                  _full(W1), _full(b1), _full(W2), _full(b2)],
        out_specs=pl.BlockSpec((R, 4), lambda i: (i, 0)),
        out_shape=jax.ShapeDtypeStruct((_N, 4), _f32),
    )(h, W1, b1, W2, b2)


def kernel(x, edge_index, edge_attr, enc_W, enc_b, enc_g, enc_bt, ee_W, ee_b,
           eW1, eb1, eg1, ebt1, eW2, eb2, eg2, ebt2,
           nW1, nb1, ng1, nbt1, nW2, nb2, ng2, nbt2,
           dW1, db1, dW2, db2):
    row3 = edge_index[0].reshape(_NG, _GROUP, _CHUNK)
    col3 = edge_index[1].reshape(_NG, _GROUP, _CHUNK)
    x8 = jnp.pad(x, ((0, 0), (0, 1)))
    W8 = jnp.pad(enc_W, ((0, 1), (0, 0)))

    cnt_raw = _sc_count(col3)
    cnt = cnt_raw.reshape(_NC, _CROWS * 16)[:, :_N].reshape(_NC, _N, 1)

    h = _tc_encode(x8, W8, enc_b, enc_g, enc_bt)
    eW1hb = eW1[:, :2 * _H, :]
    eW1e = eW1[:, 2 * _H:, :]
    e = _tc_edge_encode(edge_attr, ee_W, ee_b)

    L = eW1.shape[0]
    for i in range(L):
        ghr, ghc = _sc_gather(h, row3, col3)
        e = _tc_edge_mlp(ghr, ghc, e, eW1hb[i], eW1e[i], eb1[i], eg1[i],
                         ebt1[i], eW2[i], eb2[i], eg2[i], ebt2[i])
        agg = _sc_scatter(e, col3)
        h = _tc_node_mlp(h, agg, cnt, nW1[i], nb1[i], ng1[i],
                         nbt1[i], nW2[i], nb2[i], ng2[i], nbt2[i])

    return _tc_decode(h, dW1, db1, dW2, db2)


# speedup vs baseline: 1.1196x; 1.0295x over previous
import functools

import jax
import jax.numpy as jnp
from jax import lax
from jax.experimental import pallas as pl
from jax.experimental.pallas import tpu as pltpu
from jax.experimental.pallas import tpu_sc as plsc

_N = 50000
_E = 800000
_H = 64

_NC = 2
_NS = 16
_NW = _NC * _NS

_CHUNK = 128
_GROUP = 5
_GE = _CHUNK * _GROUP
_NR = _E // _CHUNK
_NG = _E // _GE

_HF = _H // _NC
_TROWS = _N // _NS
_CROWS = 3200
_NCH = 2

_f32 = jnp.float32
_bf16 = jnp.bfloat16
_i32 = jnp.int32


def _mesh():
    return plsc.VectorSubcoreMesh(core_axis_name="c", subcore_axis_name="s")


def _sc_gather(h, row2d, col2d):
    ngc = row2d.shape[0]
    ec = ngc * _GE

    @functools.partial(
        pl.kernel,
        out_type=(
            jax.ShapeDtypeStruct((ec, _H), _f32),
            jax.ShapeDtypeStruct((ec, _H), _f32),
        ),
        mesh=_mesh(),
        compiler_params=pltpu.CompilerParams(use_tc_tiling_on_sc=False, needs_layout_passes=False),
        scratch_types=[
            pltpu.VMEM((_GROUP, _CHUNK), _i32),
            pltpu.VMEM((_GROUP, _CHUNK), _i32),
            pltpu.VMEM((_GE, _H), _f32),
            pltpu.VMEM((_GE, _H), _f32),
            pltpu.SemaphoreType.DMA,
            pltpu.SemaphoreType.DMA,
            pltpu.SemaphoreType.DMA,
        ],
    )
    def k(h_hbm, row_hbm, col_hbm, outr, outc, idxA, idxB, bufA, bufB,
          semA, semB, semS):
        c = lax.axis_index("c")
        s = lax.axis_index("s")
        base, rem = divmod(ngc, _NS)
        npairs = base // 2
        ng = base + jnp.where(s < rem, 1, 0)

        def stream(arr_hbm, out_hbm):
            def pair(jj, carry):
                ga = s + (2 * jj) * _NS
                gb = s + (2 * jj + 1) * _NS
                pltpu.sync_copy(arr_hbm.at[ga], idxA)
                dA = [pltpu.async_copy(
                    h_hbm.at[idxA.at[t]], bufA.at[pl.ds(t * _CHUNK, _CHUNK)],
                    semA) for t in range(_GROUP)]
                pltpu.sync_copy(arr_hbm.at[gb], idxB)
                dB = [pltpu.async_copy(
                    h_hbm.at[idxB.at[t]], bufB.at[pl.ds(t * _CHUNK, _CHUNK)],
                    semB) for t in range(_GROUP)]
                for d in dA:
                    d.wait()
                dS = pltpu.async_copy(bufA, out_hbm.at[pl.ds(ga * _GE, _GE)], semS)
                for d in dB:
                    d.wait()
                pltpu.sync_copy(bufB, out_hbm.at[pl.ds(gb * _GE, _GE)])
                dS.wait()
                return carry

            lax.fori_loop(0, npairs, pair, 0)

            def tail(j, carry):
                g = s + j * _NS
                pltpu.sync_copy(arr_hbm.at[g], idxA)
                ds = [pltpu.async_copy(
                    h_hbm.at[idxA.at[t]], bufA.at[pl.ds(t * _CHUNK, _CHUNK)],
                    semA) for t in range(_GROUP)]
                for d in ds:
                    d.wait()
                pltpu.sync_copy(bufA, out_hbm.at[pl.ds(g * _GE, _GE)])
                return carry

            lax.fori_loop(2 * npairs, ng, tail, 0)

        @pl.when(c == 0)
        def _():
            stream(row_hbm, outr)

        @pl.when(c == 1)
        def _():
            stream(col_hbm, outc)

    return k(h, row2d, col2d)


def _sc_scatter(e, col2d):
    QF = 16
    ngc = col2d.shape[0]

    @functools.partial(
        pl.kernel,
        out_type=jax.ShapeDtypeStruct((_N, _H), _f32),
        mesh=_mesh(),
        compiler_params=pltpu.CompilerParams(use_tc_tiling_on_sc=False, needs_layout_passes=False),
        scratch_types=[
            pltpu.VMEM((_GROUP, _CHUNK), _i32),
            pltpu.VMEM((_GROUP, _CHUNK), _i32),
            pltpu.VMEM((_GE, QF), _f32),
            pltpu.VMEM((_GE, QF), _f32),
            pltpu.VMEM((1000, QF), _f32),
            pltpu.VMEM((1000, QF), _f32),
            pltpu.VMEM_SHARED((_N, QF), _f32),
            pltpu.SemaphoreType.DMA,
            pltpu.SemaphoreType.DMA,
            pltpu.SemaphoreType.DMA,
        ],
    )
    def k(e_hbm, col_hbm, out_hbm, idxA, idxB, ebA, ebB, bounce, zbuf, acc,
          semA, semB, semU):
        c = lax.axis_index("c")
        s = lax.axis_index("s")
        z = jnp.zeros((16,), _f32)

        def zb(i, carry):
            zbuf[i, :] = z
            return carry

        lax.fori_loop(0, 1000, zb, 0)

        nz = (_N // 1000 - s + _NS - 1) // _NS

        def zc(j, carry):
            pltpu.sync_copy(zbuf, acc.at[pl.ds((s + j * _NS) * 1000, 1000)])
            return carry

        lax.fori_loop(0, nz, zc, 0)

        base, rem = divmod(ngc, _NS)
        npairs = base // 2
        ng = base + jnp.where(s < rem, 1, 0)

        for p in range(2):
            f0 = (c + 2 * p) * QF
            plsc.subcore_barrier()

            def prefA(g):
                pltpu.async_copy(col_hbm.at[g], idxA, semA)
                pltpu.async_copy(
                    e_hbm.at[pl.ds(g * _GE, _GE), pl.ds(f0, QF)], ebA, semA)

            def waitA(g):
                pltpu.make_async_copy(col_hbm.at[g], idxA, semA).wait()
                pltpu.make_async_copy(
                    e_hbm.at[pl.ds(g * _GE, _GE), pl.ds(f0, QF)], ebA,
                    semA).wait()

            prefA(s)

            def pair(jj, carry):
                ga = s + (2 * jj) * _NS
                gb = ga + _NS
                waitA(ga)
                pltpu.async_copy(col_hbm.at[gb], idxB, semB)
                pltpu.async_copy(
                    e_hbm.at[pl.ds(gb * _GE, _GE), pl.ds(f0, QF)], ebB, semB)
                dU = [pltpu.async_copy(
                    ebA.at[pl.ds(t * _CHUNK, _CHUNK)], acc.at[idxA.at[t]],
                    semU, add=True) for t in range(_GROUP)]
                for d in dU:
                    d.wait()

                @pl.when(jj < npairs - 1)
                def _():
                    prefA(ga + 2 * _NS)

                pltpu.make_async_copy(col_hbm.at[gb], idxB, semB).wait()
                pltpu.make_async_copy(
                    e_hbm.at[pl.ds(gb * _GE, _GE), pl.ds(f0, QF)], ebB,
                    semB).wait()
                dV = [pltpu.async_copy(
                    ebB.at[pl.ds(t * _CHUNK, _CHUNK)], acc.at[idxB.at[t]],
                    semU, add=True) for t in range(_GROUP)]
                for d in dV:
                    d.wait()
                return carry

            lax.fori_loop(0, npairs, pair, 0)

            def tailS(j, carry):
                ga = s + j * _NS
                pltpu.sync_copy(col_hbm.at[ga], idxA)
                pltpu.sync_copy(
                    e_hbm.at[pl.ds(ga * _GE, _GE), pl.ds(f0, QF)], ebA)
                ds = [pltpu.async_copy(
                    ebA.at[pl.ds(t * _CHUNK, _CHUNK)], acc.at[idxA.at[t]],
                    semU, add=True) for t in range(_GROUP)]
                for d in ds:
                    d.wait()
                return carry

            lax.fori_loop(2 * npairs, ng, tailS, 0)

            plsc.subcore_barrier()

            def co(j, carry):
                r0 = (s + j * _NS) * 1000
                pltpu.sync_copy(acc.at[pl.ds(r0, 1000)], bounce)
                if p == 0:
                    pltpu.sync_copy(zbuf, acc.at[pl.ds(r0, 1000)])
                pltpu.sync_copy(bounce, out_hbm.at[pl.ds(r0, 1000), pl.ds(f0, QF)])
                return carry

            lax.fori_loop(0, nz, co, 0)

    return k(e, col2d)


def _sc_count(col2d):
    @functools.partial(
        pl.kernel,
        out_type=jax.ShapeDtypeStruct((_NC, _CROWS, 16), _f32),
        mesh=_mesh(),
        compiler_params=pltpu.CompilerParams(use_tc_tiling_on_sc=False, needs_layout_passes=False),
        scratch_types=[
            pltpu.VMEM((_CROWS, 16), _f32),
            pltpu.VMEM((_GROUP, _CHUNK), _i32),
            pltpu.VMEM((25, 128), _i32),
            pltpu.VMEM((200, 16), _f32),
            pltpu.VMEM_SHARED((_CROWS, 16), _f32),
        ],
    )
    def k(col_hbm, out_hbm, local, idxs, iotaref, bounce, acc):
        c = lax.axis_index("c")
        s = lax.axis_index("s")
        w = s * _NC + c
        z = jnp.zeros((16,), _f32)
        ones = jnp.ones((16,), _f32)
        ar = jnp.arange(16, dtype=_i32)

        def z1(i, carry):
            local[i, :] = z
            return carry

        lax.fori_loop(0, _CROWS, z1, 0)

        def z2(i, carry):
            bounce[i, :] = z
            return carry

        lax.fori_loop(0, 200, z2, 0)
        pltpu.sync_copy(bounce, acc.at[pl.ds(s * 200, 200)])

        def bi(j, carry):
            for t in range(8):
                iotaref[j, pl.ds(t * 16, 16)] = ar + (j * 128 + t * 16)
            return carry

        lax.fori_loop(0, 25, bi, 0)
        plsc.subcore_barrier()

        base, rem = divmod(_NG, _NW)
        ngr = base + jnp.where(w < rem, 1, 0)

        def body(j, carry):
            g = w + j * _NW
            pltpu.sync_copy(col_hbm.at[g], idxs)
            for t in range(_GROUP):
                for q in range(_CHUNK // 16):
                    iv = idxs[t, pl.ds(q * 16, 16)]
                    rr = lax.shift_right_logical(iv, 4)
                    cc = lax.bitwise_and(iv, 15)
                    plsc.addupdate_scatter(local, [rr, cc], ones)
            return carry

        lax.fori_loop(0, ngr, body, 0)

        def mg(j, carry):
            pltpu.sync_copy(
                local.at[pl.ds(j * 128, 128)], acc.at[iotaref.at[j]], add=True)
            return carry

        lax.fori_loop(0, 25, mg, 0)
        plsc.subcore_barrier()

        pltpu.sync_copy(acc.at[pl.ds(s * 200, 200)], bounce)
        pltpu.sync_copy(bounce, out_hbm.at[c, pl.ds(s * 200, 200)])

    return k(col2d)


def _gelu(x):
    return x * 0.5 * (1.0 + lax.erf(x * 0.7071067811865476))


def _lnorm(x, g, b):
    n = x.shape[-1]
    sx = jnp.sum(x, axis=-1, keepdims=True)
    sxx = jnp.sum(x * x, axis=-1, keepdims=True)
    m = sx * (1.0 / n)
    v = sxx * (1.0 / n) - m * m
    r = lax.rsqrt(v + 1e-5)
    return (x - m) * (r * g) + b


def _full(a):
    return pl.BlockSpec(a.shape, lambda i: (0,) * a.ndim)


def _tc_encode(x8, W8, b, g, bt):
    R = 1000

    def body(x_r, W_r, b_r, g_r, bt_r, out_r):
        t = jnp.dot(x_r[...], W_r[...], preferred_element_type=_f32) + b_r[...]
        out_r[...] = _gelu(_lnorm(t, g_r[...], bt_r[...]))

    return pl.pallas_call(
        body,
        grid=(_N // R,),
        in_specs=[pl.BlockSpec((R, 8), lambda i: (i, 0)),
                  _full(W8), _full(b), _full(g), _full(bt)],
        out_specs=pl.BlockSpec((R, _H), lambda i: (i, 0)),
        out_shape=jax.ShapeDtypeStruct((_N, _H), _f32),
    )(x8, W8, b, g, bt)


def _tc_edge_encode(ea, W, b):
    R = 1600

    def body(a_r, W_r, b_r, out_r):
        out_r[...] = jnp.dot(a_r[...], W_r[...], preferred_element_type=_f32) + b_r[...]

    return pl.pallas_call(
        body,
        grid=(_E // R,),
        in_specs=[pl.BlockSpec((R, 8), lambda i: (i, 0)), _full(W), _full(b)],
        out_specs=pl.BlockSpec((R, _H), lambda i: (i, 0)),
        out_shape=jax.ShapeDtypeStruct((_E, _H), _f32),
    )(ea, W, b)


def _tc_edge_mlp(ghr, ghc, e, W1hb, W1e, b1, g1, bt1, W2, b2, g2, bt2):
    R = 1600

    def body(ghr_r, ghc_r, e_r, W1hb_r, W1e_r, b1_r, g1_r, bt1_r, W2_r, b2_r,
             g2_r, bt2_r, out_r):
        gin = jnp.concatenate([ghr_r[...], ghc_r[...]], axis=1)
        t = (jnp.dot(gin, W1hb_r[...], preferred_element_type=_f32)
             + jnp.dot(e_r[...], W1e_r[...], preferred_element_type=_f32)
             + b1_r[...])
        t = _gelu(_lnorm(t, g1_r[...], bt1_r[...]))
        t = jnp.dot(t, W2_r[...], preferred_element_type=_f32) + b2_r[...]
        t = _lnorm(t, g2_r[...], bt2_r[...])
        out_r[...] = e_r[...] + t

    ec = e.shape[0]
    blk = pl.BlockSpec((R, _H), lambda i: (i, 0))
    return pl.pallas_call(
        body,
        grid=(ec // R,),
        in_specs=[blk, blk, blk, _full(W1hb), _full(W1e), _full(b1),
                  _full(g1), _full(bt1),
                  _full(W2), _full(b2), _full(g2), _full(bt2)],
        out_specs=blk,
        out_shape=jax.ShapeDtypeStruct((ec, _H), _f32),
    )(ghr, ghc, e, W1hb, W1e, b1, g1, bt1, W2, b2, g2, bt2)


def _tc_node_mlp(h, aggs, cnt, W1, b1, g1, bt1, W2, b2, g2, bt2):
    R = 1000
    nagg = len(aggs)

    def body(h_r, *rest):
        agg_rs = rest[:nagg]
        (cnt_r, W1_r, b1_r, g1_r, bt1_r, W2_r, b2_r, g2_r, bt2_r,
         out_r) = rest[nagg:]
        csum = cnt_r[0] + cnt_r[1]
        inv = 1.0 / jnp.maximum(csum, 1.0)
        agg = agg_rs[0][...]
        for a in agg_rs[1:]:
            agg = agg + a[...]
        nin = jnp.concatenate([h_r[...], agg * inv], axis=1)
        u = jnp.dot(nin, W1_r[...], preferred_element_type=_f32) + b1_r[...]
        u = _gelu(_lnorm(u, g1_r[...], bt1_r[...]))
        u = jnp.dot(u, W2_r[...], preferred_element_type=_f32) + b2_r[...]
        u = _lnorm(u, g2_r[...], bt2_r[...])
        out_r[...] = h_r[...] + u

    blk = pl.BlockSpec((R, _H), lambda i: (i, 0))
    return pl.pallas_call(
        body,
        grid=(_N // R,),
        in_specs=[blk] * (1 + nagg)
        + [pl.BlockSpec((2, R, 1), lambda i: (0, i, 0)),
           _full(W1), _full(b1), _full(g1), _full(bt1),
           _full(W2), _full(b2), _full(g2), _full(bt2)],
        out_specs=blk,
        out_shape=jax.ShapeDtypeStruct((_N, _H), _f32),
    )(h, *aggs, cnt, W1, b1, g1, bt1, W2, b2, g2, bt2)


def _tc_decode(h, W1, b1, W2, b2):
    R = 1000

    def body(h_r, W1_r, b1_r, W2_r, b2_r, out_r):
        t = _gelu(jnp.dot(h_r[...], W1_r[...], preferred_element_type=_f32)
                  + b1_r[...])
        out_r[...] = jnp.dot(t, W2_r[...], preferred_element_type=_f32) + b2_r[...]

    return pl.pallas_call(
        body,
        grid=(_N // R,),
        in_specs=[pl.BlockSpec((R, _H), lambda i: (i, 0)),
                  _full(W1), _full(b1), _full(W2), _full(b2)],
        out_specs=pl.BlockSpec((R, 4), lambda i: (i, 0)),
        out_shape=jax.ShapeDtypeStruct((_N, 4), _f32),
    )(h, W1, b1, W2, b2)


def kernel(x, edge_index, edge_attr, enc_W, enc_b, enc_g, enc_bt, ee_W, ee_b,
           eW1, eb1, eg1, ebt1, eW2, eb2, eg2, ebt2,
           nW1, nb1, ng1, nbt1, nW2, nb2, ng2, nbt2,
           dW1, db1, dW2, db2):
    row3 = edge_index[0].reshape(_NG, _GROUP, _CHUNK)
    col3 = edge_index[1].reshape(_NG, _GROUP, _CHUNK)
    x8 = jnp.pad(x, ((0, 0), (0, 1)))
    W8 = jnp.pad(enc_W, ((0, 1), (0, 0)))

    cnt_raw = _sc_count(col3)
    cnt = cnt_raw.reshape(_NC, _CROWS * 16)[:, :_N].reshape(_NC, _N, 1)

    h = _tc_encode(x8, W8, enc_b, enc_g, enc_bt)
    eW1hb = eW1[:, :2 * _H, :]
    eW1e = eW1[:, 2 * _H:, :]
    e = _tc_edge_encode(edge_attr, ee_W, ee_b)

    L = eW1.shape[0]
    ngc = _NG // _NCH
    ec = ngc * _GE
    rowC = [row3[k * ngc:(k + 1) * ngc] for k in range(_NCH)]
    colC = [col3[k * ngc:(k + 1) * ngc] for k in range(_NCH)]
    eC = [e[k * ec:(k + 1) * ec] for k in range(_NCH)]

    for i in range(L):
        gh = [_sc_gather(h, rowC[k], colC[k]) for k in range(_NCH)]
        aggs = []
        for k in range(_NCH):
            eC[k] = _tc_edge_mlp(gh[k][0], gh[k][1], eC[k], eW1hb[i], eW1e[i],
                                 eb1[i], eg1[i], ebt1[i], eW2[i], eb2[i],
                                 eg2[i], ebt2[i])
            aggs.append(_sc_scatter(eC[k], colC[k]))
        h = _tc_node_mlp(h, aggs, cnt, nW1[i], nb1[i], ng1[i],
                         nbt1[i], nW2[i], nb2[i], ng2[i], nbt2[i])

    return _tc_decode(h, dW1, db1, dW2, db2)


# speedup vs baseline: 1.1250x; 1.0048x over previous
import functools

import jax
import jax.numpy as jnp
from jax import lax
from jax.experimental import pallas as pl
from jax.experimental.pallas import tpu as pltpu
from jax.experimental.pallas import tpu_sc as plsc

_N = 50000
_E = 800000
_H = 64

_NC = 2
_NS = 16
_NW = _NC * _NS

_CHUNK = 128
_GROUP = 5
_GE = _CHUNK * _GROUP
_NR = _E // _CHUNK
_NG = _E // _GE

_HF = _H // _NC
_TROWS = _N // _NS
_CROWS = 3200
_NCH = 5

_f32 = jnp.float32
_bf16 = jnp.bfloat16
_i32 = jnp.int32


def _mesh():
    return plsc.VectorSubcoreMesh(core_axis_name="c", subcore_axis_name="s")


def _sc_gather(h, row2d, col2d):
    ngc = row2d.shape[0]
    ec = ngc * _GE

    @functools.partial(
        pl.kernel,
        out_type=(
            jax.ShapeDtypeStruct((ec, _H), _f32),
            jax.ShapeDtypeStruct((ec, _H), _f32),
        ),
        mesh=_mesh(),
        compiler_params=pltpu.CompilerParams(use_tc_tiling_on_sc=False, needs_layout_passes=False),
        scratch_types=[
            pltpu.VMEM((_GROUP, _CHUNK), _i32),
            pltpu.VMEM((_GROUP, _CHUNK), _i32),
            pltpu.VMEM((_GE, _H), _f32),
            pltpu.VMEM((_GE, _H), _f32),
            pltpu.SemaphoreType.DMA,
            pltpu.SemaphoreType.DMA,
            pltpu.SemaphoreType.DMA,
        ],
    )
    def k(h_hbm, row_hbm, col_hbm, outr, outc, idxA, idxB, bufA, bufB,
          semA, semB, semS):
        c = lax.axis_index("c")
        s = lax.axis_index("s")
        base, rem = divmod(ngc, _NS)
        npairs = base // 2
        ng = base + jnp.where(s < rem, 1, 0)

        def stream(arr_hbm, out_hbm):
            def pair(jj, carry):
                ga = s + (2 * jj) * _NS
                gb = s + (2 * jj + 1) * _NS
                pltpu.sync_copy(arr_hbm.at[ga], idxA)
                dA = [pltpu.async_copy(
                    h_hbm.at[idxA.at[t]], bufA.at[pl.ds(t * _CHUNK, _CHUNK)],
                    semA) for t in range(_GROUP)]
                pltpu.sync_copy(arr_hbm.at[gb], idxB)
                dB = [pltpu.async_copy(
                    h_hbm.at[idxB.at[t]], bufB.at[pl.ds(t * _CHUNK, _CHUNK)],
                    semB) for t in range(_GROUP)]
                for d in dA:
                    d.wait()
                dS = pltpu.async_copy(bufA, out_hbm.at[pl.ds(ga * _GE, _GE)], semS)
                for d in dB:
                    d.wait()
                pltpu.sync_copy(bufB, out_hbm.at[pl.ds(gb * _GE, _GE)])
                dS.wait()
                return carry

            lax.fori_loop(0, npairs, pair, 0)

            def tail(j, carry):
                g = s + j * _NS
                pltpu.sync_copy(arr_hbm.at[g], idxA)
                ds = [pltpu.async_copy(
                    h_hbm.at[idxA.at[t]], bufA.at[pl.ds(t * _CHUNK, _CHUNK)],
                    semA) for t in range(_GROUP)]
                for d in ds:
                    d.wait()
                pltpu.sync_copy(bufA, out_hbm.at[pl.ds(g * _GE, _GE)])
                return carry

            lax.fori_loop(2 * npairs, ng, tail, 0)

        @pl.when(c == 0)
        def _():
            stream(row_hbm, outr)

        @pl.when(c == 1)
        def _():
            stream(col_hbm, outc)

    return k(h, row2d, col2d)


def _sc_scatter(e, col2d):
    QF = 16
    ngc = col2d.shape[0]

    @functools.partial(
        pl.kernel,
        out_type=jax.ShapeDtypeStruct((_N, _H), _f32),
        mesh=_mesh(),
        compiler_params=pltpu.CompilerParams(use_tc_tiling_on_sc=False, needs_layout_passes=False),
        scratch_types=[
            pltpu.VMEM((_GROUP, _CHUNK), _i32),
            pltpu.VMEM((_GROUP, _CHUNK), _i32),
            pltpu.VMEM((_GE, QF), _f32),
            pltpu.VMEM((_GE, QF), _f32),
            pltpu.VMEM((1000, QF), _f32),
            pltpu.VMEM((1000, QF), _f32),
            pltpu.VMEM_SHARED((_N, QF), _f32),
            pltpu.SemaphoreType.DMA,
            pltpu.SemaphoreType.DMA,
            pltpu.SemaphoreType.DMA,
        ],
    )
    def k(e_hbm, col_hbm, out_hbm, idxA, idxB, ebA, ebB, bounce, zbuf, acc,
          semA, semB, semU):
        c = lax.axis_index("c")
        s = lax.axis_index("s")
        z = jnp.zeros((16,), _f32)

        def zb(i, carry):
            zbuf[i, :] = z
            return carry

        lax.fori_loop(0, 1000, zb, 0)

        nz = (_N // 1000 - s + _NS - 1) // _NS

        def zc(j, carry):
            pltpu.sync_copy(zbuf, acc.at[pl.ds((s + j * _NS) * 1000, 1000)])
            return carry

        lax.fori_loop(0, nz, zc, 0)

        base, rem = divmod(ngc, _NS)
        npairs = base // 2
        ng = base + jnp.where(s < rem, 1, 0)

        for p in range(2):
            f0 = (c + 2 * p) * QF
            plsc.subcore_barrier()

            def prefA(g):
                pltpu.async_copy(col_hbm.at[g], idxA, semA)
                pltpu.async_copy(
                    e_hbm.at[pl.ds(g * _GE, _GE), pl.ds(f0, QF)], ebA, semA)

            def waitA(g):
                pltpu.make_async_copy(col_hbm.at[g], idxA, semA).wait()
                pltpu.make_async_copy(
                    e_hbm.at[pl.ds(g * _GE, _GE), pl.ds(f0, QF)], ebA,
                    semA).wait()

            prefA(s)

            def pair(jj, carry):
                ga = s + (2 * jj) * _NS
                gb = ga + _NS
                waitA(ga)
                pltpu.async_copy(col_hbm.at[gb], idxB, semB)
                pltpu.async_copy(
                    e_hbm.at[pl.ds(gb * _GE, _GE), pl.ds(f0, QF)], ebB, semB)
                dU = [pltpu.async_copy(
                    ebA.at[pl.ds(t * _CHUNK, _CHUNK)], acc.at[idxA.at[t]],
                    semU, add=True) for t in range(_GROUP)]
                for d in dU:
                    d.wait()

                @pl.when(jj < npairs - 1)
                def _():
                    prefA(ga + 2 * _NS)

                pltpu.make_async_copy(col_hbm.at[gb], idxB, semB).wait()
                pltpu.make_async_copy(
                    e_hbm.at[pl.ds(gb * _GE, _GE), pl.ds(f0, QF)], ebB,
                    semB).wait()
                dV = [pltpu.async_copy(
                    ebB.at[pl.ds(t * _CHUNK, _CHUNK)], acc.at[idxB.at[t]],
                    semU, add=True) for t in range(_GROUP)]
                for d in dV:
                    d.wait()
                return carry

            lax.fori_loop(0, npairs, pair, 0)

            def tailS(j, carry):
                ga = s + j * _NS
                pltpu.sync_copy(col_hbm.at[ga], idxA)
                pltpu.sync_copy(
                    e_hbm.at[pl.ds(ga * _GE, _GE), pl.ds(f0, QF)], ebA)
                ds = [pltpu.async_copy(
                    ebA.at[pl.ds(t * _CHUNK, _CHUNK)], acc.at[idxA.at[t]],
                    semU, add=True) for t in range(_GROUP)]
                for d in ds:
                    d.wait()
                return carry

            lax.fori_loop(2 * npairs, ng, tailS, 0)

            plsc.subcore_barrier()

            def co(j, carry):
                r0 = (s + j * _NS) * 1000
                pltpu.sync_copy(acc.at[pl.ds(r0, 1000)], bounce)
                if p == 0:
                    pltpu.sync_copy(zbuf, acc.at[pl.ds(r0, 1000)])
                pltpu.sync_copy(bounce, out_hbm.at[pl.ds(r0, 1000), pl.ds(f0, QF)])
                return carry

            lax.fori_loop(0, nz, co, 0)

    return k(e, col2d)


def _sc_count(col2d):
    @functools.partial(
        pl.kernel,
        out_type=jax.ShapeDtypeStruct((_NC, _CROWS, 16), _f32),
        mesh=_mesh(),
        compiler_params=pltpu.CompilerParams(use_tc_tiling_on_sc=False, needs_layout_passes=False),
        scratch_types=[
            pltpu.VMEM((_CROWS, 16), _f32),
            pltpu.VMEM((_GROUP, _CHUNK), _i32),
            pltpu.VMEM((25, 128), _i32),
            pltpu.VMEM((200, 16), _f32),
            pltpu.VMEM_SHARED((_CROWS, 16), _f32),
        ],
    )
    def k(col_hbm, out_hbm, local, idxs, iotaref, bounce, acc):
        c = lax.axis_index("c")
        s = lax.axis_index("s")
        w = s * _NC + c
        z = jnp.zeros((16,), _f32)
        ones = jnp.ones((16,), _f32)
        ar = jnp.arange(16, dtype=_i32)

        def z1(i, carry):
            local[i, :] = z
            return carry

        lax.fori_loop(0, _CROWS, z1, 0)

        def z2(i, carry):
            bounce[i, :] = z
            return carry

        lax.fori_loop(0, 200, z2, 0)
        pltpu.sync_copy(bounce, acc.at[pl.ds(s * 200, 200)])

        def bi(j, carry):
            for t in range(8):
                iotaref[j, pl.ds(t * 16, 16)] = ar + (j * 128 + t * 16)
            return carry

        lax.fori_loop(0, 25, bi, 0)
        plsc.subcore_barrier()

        base, rem = divmod(_NG, _NW)
        ngr = base + jnp.where(w < rem, 1, 0)

        def body(j, carry):
            g = w + j * _NW
            pltpu.sync_copy(col_hbm.at[g], idxs)
            for t in range(_GROUP):
                for q in range(_CHUNK // 16):
                    iv = idxs[t, pl.ds(q * 16, 16)]
                    rr = lax.shift_right_logical(iv, 4)
                    cc = lax.bitwise_and(iv, 15)
                    plsc.addupdate_scatter(local, [rr, cc], ones)
            return carry

        lax.fori_loop(0, ngr, body, 0)

        def mg(j, carry):
            pltpu.sync_copy(
                local.at[pl.ds(j * 128, 128)], acc.at[iotaref.at[j]], add=True)
            return carry

        lax.fori_loop(0, 25, mg, 0)
        plsc.subcore_barrier()

        pltpu.sync_copy(acc.at[pl.ds(s * 200, 200)], bounce)
        pltpu.sync_copy(bounce, out_hbm.at[c, pl.ds(s * 200, 200)])

    return k(col2d)


def _gelu(x):
    return x * 0.5 * (1.0 + lax.erf(x * 0.7071067811865476))


def _lnorm(x, g, b):
    n = x.shape[-1]
    sx = jnp.sum(x, axis=-1, keepdims=True)
    sxx = jnp.sum(x * x, axis=-1, keepdims=True)
    m = sx * (1.0 / n)
    v = sxx * (1.0 / n) - m * m
    r = lax.rsqrt(v + 1e-5)
    return (x - m) * (r * g) + b


def _full(a):
    return pl.BlockSpec(a.shape, lambda i: (0,) * a.ndim)


def _tc_encode(x8, W8, b, g, bt):
    R = 1000

    def body(x_r, W_r, b_r, g_r, bt_r, out_r):
        t = jnp.dot(x_r[...], W_r[...], preferred_element_type=_f32) + b_r[...]
        out_r[...] = _gelu(_lnorm(t, g_r[...], bt_r[...]))

    return pl.pallas_call(
        body,
        grid=(_N // R,),
        in_specs=[pl.BlockSpec((R, 8), lambda i: (i, 0)),
                  _full(W8), _full(b), _full(g), _full(bt)],
        out_specs=pl.BlockSpec((R, _H), lambda i: (i, 0)),
        out_shape=jax.ShapeDtypeStruct((_N, _H), _f32),
    )(x8, W8, b, g, bt)


def _tc_edge_encode(ea, W, b):
    R = 1600

    def body(a_r, W_r, b_r, out_r):
        out_r[...] = jnp.dot(a_r[...], W_r[...], preferred_element_type=_f32) + b_r[...]

    return pl.pallas_call(
        body,
        grid=(_E // R,),
        in_specs=[pl.BlockSpec((R, 8), lambda i: (i, 0)), _full(W), _full(b)],
        out_specs=pl.BlockSpec((R, _H), lambda i: (i, 0)),
        out_shape=jax.ShapeDtypeStruct((_E, _H), _f32),
    )(ea, W, b)


def _tc_edge_mlp(ghr, ghc, e, W1hb, W1e, b1, g1, bt1, W2, b2, g2, bt2):
    R = 1600

    def body(ghr_r, ghc_r, e_r, W1hb_r, W1e_r, b1_r, g1_r, bt1_r, W2_r, b2_r,
             g2_r, bt2_r, out_r):
        gin = jnp.concatenate([ghr_r[...], ghc_r[...]], axis=1)
        t = (jnp.dot(gin, W1hb_r[...], preferred_element_type=_f32)
             + jnp.dot(e_r[...], W1e_r[...], preferred_element_type=_f32)
             + b1_r[...])
        t = _gelu(_lnorm(t, g1_r[...], bt1_r[...]))
        t = jnp.dot(t, W2_r[...], preferred_element_type=_f32) + b2_r[...]
        t = _lnorm(t, g2_r[...], bt2_r[...])
        out_r[...] = e_r[...] + t

    ec = e.shape[0]
    blk = pl.BlockSpec((R, _H), lambda i: (i, 0))
    return pl.pallas_call(
        body,
        grid=(ec // R,),
        in_specs=[blk, blk, blk, _full(W1hb), _full(W1e), _full(b1),
                  _full(g1), _full(bt1),
                  _full(W2), _full(b2), _full(g2), _full(bt2)],
        out_specs=blk,
        out_shape=jax.ShapeDtypeStruct((ec, _H), _f32),
    )(ghr, ghc, e, W1hb, W1e, b1, g1, bt1, W2, b2, g2, bt2)


def _tc_node_mlp(h, aggs, cnt, W1, b1, g1, bt1, W2, b2, g2, bt2):
    R = 1000
    nagg = len(aggs)

    def body(h_r, *rest):
        agg_rs = rest[:nagg]
        (cnt_r, W1_r, b1_r, g1_r, bt1_r, W2_r, b2_r, g2_r, bt2_r,
         out_r) = rest[nagg:]
        csum = cnt_r[0] + cnt_r[1]
        inv = 1.0 / jnp.maximum(csum, 1.0)
        agg = agg_rs[0][...]
        for a in agg_rs[1:]:
            agg = agg + a[...]
        nin = jnp.concatenate([h_r[...], agg * inv], axis=1)
        u = jnp.dot(nin, W1_r[...], preferred_element_type=_f32) + b1_r[...]
        u = _gelu(_lnorm(u, g1_r[...], bt1_r[...]))
        u = jnp.dot(u, W2_r[...], preferred_element_type=_f32) + b2_r[...]
        u = _lnorm(u, g2_r[...], bt2_r[...])
        out_r[...] = h_r[...] + u

    blk = pl.BlockSpec((R, _H), lambda i: (i, 0))
    return pl.pallas_call(
        body,
        grid=(_N // R,),
        in_specs=[blk] * (1 + nagg)
        + [pl.BlockSpec((2, R, 1), lambda i: (0, i, 0)),
           _full(W1), _full(b1), _full(g1), _full(bt1),
           _full(W2), _full(b2), _full(g2), _full(bt2)],
        out_specs=blk,
        out_shape=jax.ShapeDtypeStruct((_N, _H), _f32),
    )(h, *aggs, cnt, W1, b1, g1, bt1, W2, b2, g2, bt2)


def _tc_decode(h, W1, b1, W2, b2):
    R = 1000

    def body(h_r, W1_r, b1_r, W2_r, b2_r, out_r):
        t = _gelu(jnp.dot(h_r[...], W1_r[...], preferred_element_type=_f32)
                  + b1_r[...])
        out_r[...] = jnp.dot(t, W2_r[...], preferred_element_type=_f32) + b2_r[...]

    return pl.pallas_call(
        body,
        grid=(_N // R,),
        in_specs=[pl.BlockSpec((R, _H), lambda i: (i, 0)),
                  _full(W1), _full(b1), _full(W2), _full(b2)],
        out_specs=pl.BlockSpec((R, 4), lambda i: (i, 0)),
        out_shape=jax.ShapeDtypeStruct((_N, 4), _f32),
    )(h, W1, b1, W2, b2)


def kernel(x, edge_index, edge_attr, enc_W, enc_b, enc_g, enc_bt, ee_W, ee_b,
           eW1, eb1, eg1, ebt1, eW2, eb2, eg2, ebt2,
           nW1, nb1, ng1, nbt1, nW2, nb2, ng2, nbt2,
           dW1, db1, dW2, db2):
    row3 = edge_index[0].reshape(_NG, _GROUP, _CHUNK)
    col3 = edge_index[1].reshape(_NG, _GROUP, _CHUNK)
    x8 = jnp.pad(x, ((0, 0), (0, 1)))
    W8 = jnp.pad(enc_W, ((0, 1), (0, 0)))

    cnt_raw = _sc_count(col3)
    cnt = cnt_raw.reshape(_NC, _CROWS * 16)[:, :_N].reshape(_NC, _N, 1)

    h = _tc_encode(x8, W8, enc_b, enc_g, enc_bt)
    eW1hb = eW1[:, :2 * _H, :]
    eW1e = eW1[:, 2 * _H:, :]
    e = _tc_edge_encode(edge_attr, ee_W, ee_b)

    L = eW1.shape[0]
    ngc = _NG // _NCH
    ec = ngc * _GE
    rowC = [row3[k * ngc:(k + 1) * ngc] for k in range(_NCH)]
    colC = [col3[k * ngc:(k + 1) * ngc] for k in range(_NCH)]
    eC = [e[k * ec:(k + 1) * ec] for k in range(_NCH)]

    for i in range(L):
        gh = [_sc_gather(h, rowC[k], colC[k]) for k in range(_NCH)]
        aggs = []
        for k in range(_NCH):
            eC[k] = _tc_edge_mlp(gh[k][0], gh[k][1], eC[k], eW1hb[i], eW1e[i],
                                 eb1[i], eg1[i], ebt1[i], eW2[i], eb2[i],
                                 eg2[i], ebt2[i])
            aggs.append(_sc_scatter(eC[k], colC[k]))
        h = _tc_node_mlp(h, aggs, cnt, nW1[i], nb1[i], ng1[i],
                         nbt1[i], nW2[i], nb2[i], ng2[i], nbt2[i])

    return _tc_decode(h, dW1, db1, dW2, db2)


# speedup vs baseline: 1.1877x; 1.0558x over previous
import functools

import jax
import jax.numpy as jnp
from jax import lax
from jax.experimental import pallas as pl
from jax.experimental.pallas import tpu as pltpu
from jax.experimental.pallas import tpu_sc as plsc

_N = 50000
_E = 800000
_H = 64

_NC = 2
_NS = 16
_NW = _NC * _NS

_CHUNK = 128
_GROUP = 5
_GE = _CHUNK * _GROUP
_NR = _E // _CHUNK
_NG = _E // _GE

_HF = _H // _NC
_TROWS = _N // _NS
_CROWS = 3200
_NCH = 5

_f32 = jnp.float32
_bf16 = jnp.bfloat16
_i32 = jnp.int32


def _mesh():
    return plsc.VectorSubcoreMesh(core_axis_name="c", subcore_axis_name="s")


def _sc_gather(h, row2d, col2d):
    ngc = row2d.shape[0]
    ec = ngc * _GE

    @functools.partial(
        pl.kernel,
        out_type=(
            jax.ShapeDtypeStruct((ec, _H), _f32),
            jax.ShapeDtypeStruct((ec, _H), _f32),
        ),
        mesh=_mesh(),
        compiler_params=pltpu.CompilerParams(use_tc_tiling_on_sc=False, needs_layout_passes=False),
        scratch_types=[
            pltpu.VMEM((_GROUP, _CHUNK), _i32),
            pltpu.VMEM((_GROUP, _CHUNK), _i32),
            pltpu.VMEM((_GE, _H), _f32),
            pltpu.VMEM((_GE, _H), _f32),
            pltpu.SemaphoreType.DMA,
            pltpu.SemaphoreType.DMA,
            pltpu.SemaphoreType.DMA,
        ],
    )
    def k(h_hbm, row_hbm, col_hbm, outr, outc, idxA, idxB, bufA, bufB,
          semA, semB, semS):
        c = lax.axis_index("c")
        s = lax.axis_index("s")
        base, rem = divmod(ngc, _NS)
        npairs = base // 2
        ng = base + jnp.where(s < rem, 1, 0)

        def stream(arr_hbm, out_hbm):
            def pair(jj, carry):
                ga = s + (2 * jj) * _NS
                gb = s + (2 * jj + 1) * _NS
                pltpu.sync_copy(arr_hbm.at[ga], idxA)
                dA = [pltpu.async_copy(
                    h_hbm.at[idxA.at[t]], bufA.at[pl.ds(t * _CHUNK, _CHUNK)],
                    semA) for t in range(_GROUP)]
                pltpu.sync_copy(arr_hbm.at[gb], idxB)
                dB = [pltpu.async_copy(
                    h_hbm.at[idxB.at[t]], bufB.at[pl.ds(t * _CHUNK, _CHUNK)],
                    semB) for t in range(_GROUP)]
                for d in dA:
                    d.wait()
                dS = pltpu.async_copy(bufA, out_hbm.at[pl.ds(ga * _GE, _GE)], semS)
                for d in dB:
                    d.wait()
                pltpu.sync_copy(bufB, out_hbm.at[pl.ds(gb * _GE, _GE)])
                dS.wait()
                return carry

            lax.fori_loop(0, npairs, pair, 0)

            def tail(j, carry):
                g = s + j * _NS
                pltpu.sync_copy(arr_hbm.at[g], idxA)
                ds = [pltpu.async_copy(
                    h_hbm.at[idxA.at[t]], bufA.at[pl.ds(t * _CHUNK, _CHUNK)],
                    semA) for t in range(_GROUP)]
                for d in ds:
                    d.wait()
                pltpu.sync_copy(bufA, out_hbm.at[pl.ds(g * _GE, _GE)])
                return carry

            lax.fori_loop(2 * npairs, ng, tail, 0)

        @pl.when(c == 0)
        def _():
            stream(row_hbm, outr)

        @pl.when(c == 1)
        def _():
            stream(col_hbm, outc)

    return k(h, row2d, col2d)


def _sc_scatter(e, col2d):
    QF = 16
    ngc = col2d.shape[0]

    @functools.partial(
        pl.kernel,
        out_type=jax.ShapeDtypeStruct((_N, _H), _f32),
        mesh=_mesh(),
        compiler_params=pltpu.CompilerParams(use_tc_tiling_on_sc=False, needs_layout_passes=False),
        scratch_types=[
            pltpu.VMEM((_GROUP, _CHUNK), _i32),
            pltpu.VMEM((_GROUP, _CHUNK), _i32),
            pltpu.VMEM((_GE, QF), _f32),
            pltpu.VMEM((_GE, QF), _f32),
            pltpu.VMEM((1000, QF), _f32),
            pltpu.VMEM((1000, QF), _f32),
            pltpu.VMEM_SHARED((_N, QF), _f32),
            pltpu.SemaphoreType.DMA,
            pltpu.SemaphoreType.DMA,
            pltpu.SemaphoreType.DMA,
        ],
    )
    def k(e_hbm, col_hbm, out_hbm, idxA, idxB, ebA, ebB, bounce, zbuf, acc,
          semA, semB, semU):
        c = lax.axis_index("c")
        s = lax.axis_index("s")
        z = jnp.zeros((16,), _f32)

        def zb(i, carry):
            zbuf[i, :] = z
            return carry

        lax.fori_loop(0, 1000, zb, 0)

        nz = (_N // 1000 - s + _NS - 1) // _NS

        def zc(j, carry):
            pltpu.sync_copy(zbuf, acc.at[pl.ds((s + j * _NS) * 1000, 1000)])
            return carry

        lax.fori_loop(0, nz, zc, 0)

        base, rem = divmod(ngc, _NS)
        npairs = base // 2
        ng = base + jnp.where(s < rem, 1, 0)

        for p in range(2):
            f0 = (c + 2 * p) * QF
            plsc.subcore_barrier()

            def prefA(g):
                pltpu.async_copy(col_hbm.at[g], idxA, semA)
                pltpu.async_copy(
                    e_hbm.at[pl.ds(g * _GE, _GE), pl.ds(f0, QF)], ebA, semA)

            def waitA(g):
                pltpu.make_async_copy(col_hbm.at[g], idxA, semA).wait()
                pltpu.make_async_copy(
                    e_hbm.at[pl.ds(g * _GE, _GE), pl.ds(f0, QF)], ebA,
                    semA).wait()

            prefA(s)

            def pair(jj, carry):
                ga = s + (2 * jj) * _NS
                gb = ga + _NS
                waitA(ga)
                pltpu.async_copy(col_hbm.at[gb], idxB, semB)
                pltpu.async_copy(
                    e_hbm.at[pl.ds(gb * _GE, _GE), pl.ds(f0, QF)], ebB, semB)
                dU = [pltpu.async_copy(
                    ebA.at[pl.ds(t * _CHUNK, _CHUNK)], acc.at[idxA.at[t]],
                    semU, add=True) for t in range(_GROUP)]
                for d in dU:
                    d.wait()

                @pl.when(jj < npairs - 1)
                def _():
                    prefA(ga + 2 * _NS)

                pltpu.make_async_copy(col_hbm.at[gb], idxB, semB).wait()
                pltpu.make_async_copy(
                    e_hbm.at[pl.ds(gb * _GE, _GE), pl.ds(f0, QF)], ebB,
                    semB).wait()
                dV = [pltpu.async_copy(
                    ebB.at[pl.ds(t * _CHUNK, _CHUNK)], acc.at[idxB.at[t]],
                    semU, add=True) for t in range(_GROUP)]
                for d in dV:
                    d.wait()
                return carry

            lax.fori_loop(0, npairs, pair, 0)

            def tailS(j, carry):
                ga = s + j * _NS
                pltpu.sync_copy(col_hbm.at[ga], idxA)
                pltpu.sync_copy(
                    e_hbm.at[pl.ds(ga * _GE, _GE), pl.ds(f0, QF)], ebA)
                ds = [pltpu.async_copy(
                    ebA.at[pl.ds(t * _CHUNK, _CHUNK)], acc.at[idxA.at[t]],
                    semU, add=True) for t in range(_GROUP)]
                for d in ds:
                    d.wait()
                return carry

            lax.fori_loop(2 * npairs, ng, tailS, 0)

            plsc.subcore_barrier()

            def co(j, carry):
                r0 = (s + j * _NS) * 1000
                pltpu.sync_copy(acc.at[pl.ds(r0, 1000)], bounce)
                if p == 0:
                    pltpu.sync_copy(zbuf, acc.at[pl.ds(r0, 1000)])
                pltpu.sync_copy(bounce, out_hbm.at[pl.ds(r0, 1000), pl.ds(f0, QF)])
                return carry

            lax.fori_loop(0, nz, co, 0)

    return k(e, col2d)


def _sc_count(col2d):
    @functools.partial(
        pl.kernel,
        out_type=jax.ShapeDtypeStruct((_NC, _CROWS, 16), _f32),
        mesh=_mesh(),
        compiler_params=pltpu.CompilerParams(use_tc_tiling_on_sc=False, needs_layout_passes=False),
        scratch_types=[
            pltpu.VMEM((_CROWS, 16), _f32),
            pltpu.VMEM((_GROUP, _CHUNK), _i32),
            pltpu.VMEM((25, 128), _i32),
            pltpu.VMEM((200, 16), _f32),
            pltpu.VMEM_SHARED((_CROWS, 16), _f32),
        ],
    )
    def k(col_hbm, out_hbm, local, idxs, iotaref, bounce, acc):
        c = lax.axis_index("c")
        s = lax.axis_index("s")
        w = s * _NC + c
        z = jnp.zeros((16,), _f32)
        ones = jnp.ones((16,), _f32)
        ar = jnp.arange(16, dtype=_i32)

        def z1(i, carry):
            local[i, :] = z
            return carry

        lax.fori_loop(0, _CROWS, z1, 0)

        def z2(i, carry):
            bounce[i, :] = z
            return carry

        lax.fori_loop(0, 200, z2, 0)
        pltpu.sync_copy(bounce, acc.at[pl.ds(s * 200, 200)])

        def bi(j, carry):
            for t in range(8):
                iotaref[j, pl.ds(t * 16, 16)] = ar + (j * 128 + t * 16)
            return carry

        lax.fori_loop(0, 25, bi, 0)
        plsc.subcore_barrier()

        base, rem = divmod(_NG, _NW)
        ngr = base + jnp.where(w < rem, 1, 0)

        def body(j, carry):
            g = w + j * _NW
            pltpu.sync_copy(col_hbm.at[g], idxs)
            for t in range(_GROUP):
                for q in range(_CHUNK // 16):
                    iv = idxs[t, pl.ds(q * 16, 16)]
                    rr = lax.shift_right_logical(iv, 4)
                    cc = lax.bitwise_and(iv, 15)
                    plsc.addupdate_scatter(local, [rr, cc], ones)
            return carry

        lax.fori_loop(0, ngr, body, 0)

        def mg(j, carry):
            pltpu.sync_copy(
                local.at[pl.ds(j * 128, 128)], acc.at[iotaref.at[j]], add=True)
            return carry

        lax.fori_loop(0, 25, mg, 0)
        plsc.subcore_barrier()

        pltpu.sync_copy(acc.at[pl.ds(s * 200, 200)], bounce)
        pltpu.sync_copy(bounce, out_hbm.at[c, pl.ds(s * 200, 200)])

    return k(col2d)


def _gelu(x):
    return x * 0.5 * (1.0 + lax.erf(x * 0.7071067811865476))


def _lnorm(x, g, b):
    n = x.shape[-1]
    sx = jnp.sum(x, axis=-1, keepdims=True)
    sxx = jnp.sum(x * x, axis=-1, keepdims=True)
    m = sx * (1.0 / n)
    v = sxx * (1.0 / n) - m * m
    r = lax.rsqrt(v + 1e-5)
    return (x - m) * (r * g) + b


def _full(a):
    return pl.BlockSpec(a.shape, lambda i: (0,) * a.ndim)


def _tc_encode(x8, W8, b, g, bt):
    R = 1000

    def body(x_r, W_r, b_r, g_r, bt_r, out_r):
        t = jnp.dot(x_r[...], W_r[...], preferred_element_type=_f32) + b_r[...]
        out_r[...] = _gelu(_lnorm(t, g_r[...], bt_r[...]))

    return pl.pallas_call(
        body,
        grid=(_N // R,),
        in_specs=[pl.BlockSpec((R, 8), lambda i: (i, 0)),
                  _full(W8), _full(b), _full(g), _full(bt)],
        out_specs=pl.BlockSpec((R, _H), lambda i: (i, 0)),
        out_shape=jax.ShapeDtypeStruct((_N, _H), _f32),
    )(x8, W8, b, g, bt)


def _tc_edge_encode(ea, W, b):
    R = 1600

    def body(a_r, W_r, b_r, out_r):
        out_r[...] = jnp.dot(a_r[...], W_r[...], preferred_element_type=_f32) + b_r[...]

    return pl.pallas_call(
        body,
        grid=(_E // R,),
        in_specs=[pl.BlockSpec((R, 8), lambda i: (i, 0)), _full(W), _full(b)],
        out_specs=pl.BlockSpec((R, _H), lambda i: (i, 0)),
        out_shape=jax.ShapeDtypeStruct((_E, _H), _f32),
    )(ea, W, b)


def _tc_edge_mlp(ghr, ghc, e, W1hb, W1e, b1, g1, bt1, W2, b2, g2, bt2):
    R = 3200

    def body(ghr_r, ghc_r, e_r, W1hb_r, W1e_r, b1_r, g1_r, bt1_r, W2_r, b2_r,
             g2_r, bt2_r, out_r):
        gin = jnp.concatenate([ghr_r[...], ghc_r[...]], axis=1)
        t = (jnp.dot(gin, W1hb_r[...], preferred_element_type=_f32)
             + jnp.dot(e_r[...], W1e_r[...], preferred_element_type=_f32)
             + b1_r[...])
        t = _gelu(_lnorm(t, g1_r[...], bt1_r[...]))
        t = jnp.dot(t, W2_r[...], preferred_element_type=_f32) + b2_r[...]
        t = _lnorm(t, g2_r[...], bt2_r[...])
        out_r[...] = e_r[...] + t

    ec = e.shape[0]
    blk = pl.BlockSpec((R, _H), lambda i: (i, 0))
    return pl.pallas_call(
        body,
        grid=(ec // R,),
        in_specs=[blk, blk, blk, _full(W1hb), _full(W1e), _full(b1),
                  _full(g1), _full(bt1),
                  _full(W2), _full(b2), _full(g2), _full(bt2)],
        out_specs=blk,
        out_shape=jax.ShapeDtypeStruct((ec, _H), _f32),
    )(ghr, ghc, e, W1hb, W1e, b1, g1, bt1, W2, b2, g2, bt2)


def _tc_node_mlp(h, aggs, cnt, W1, b1, g1, bt1, W2, b2, g2, bt2):
    R = 1000
    nagg = len(aggs)

    def body(h_r, *rest):
        agg_rs = rest[:nagg]
        (cnt_r, W1_r, b1_r, g1_r, bt1_r, W2_r, b2_r, g2_r, bt2_r,
         out_r) = rest[nagg:]
        csum = cnt_r[0] + cnt_r[1]
        inv = 1.0 / jnp.maximum(csum, 1.0)
        agg = agg_rs[0][...]
        for a in agg_rs[1:]:
            agg = agg + a[...]
        nin = jnp.concatenate([h_r[...], agg * inv], axis=1)
        u = jnp.dot(nin, W1_r[...], preferred_element_type=_f32) + b1_r[...]
        u = _gelu(_lnorm(u, g1_r[...], bt1_r[...]))
        u = jnp.dot(u, W2_r[...], preferred_element_type=_f32) + b2_r[...]
        u = _lnorm(u, g2_r[...], bt2_r[...])
        out_r[...] = h_r[...] + u

    blk = pl.BlockSpec((R, _H), lambda i: (i, 0))
    return pl.pallas_call(
        body,
        grid=(_N // R,),
        in_specs=[blk] * (1 + nagg)
        + [pl.BlockSpec((2, R, 1), lambda i: (0, i, 0)),
           _full(W1), _full(b1), _full(g1), _full(bt1),
           _full(W2), _full(b2), _full(g2), _full(bt2)],
        out_specs=blk,
        out_shape=jax.ShapeDtypeStruct((_N, _H), _f32),
    )(h, *aggs, cnt, W1, b1, g1, bt1, W2, b2, g2, bt2)


def _tc_decode(h, W1, b1, W2, b2):
    R = 1000

    def body(h_r, W1_r, b1_r, W2_r, b2_r, out_r):
        t = _gelu(jnp.dot(h_r[...], W1_r[...], preferred_element_type=_f32)
                  + b1_r[...])
        out_r[...] = jnp.dot(t, W2_r[...], preferred_element_type=_f32) + b2_r[...]

    return pl.pallas_call(
        body,
        grid=(_N // R,),
        in_specs=[pl.BlockSpec((R, _H), lambda i: (i, 0)),
                  _full(W1), _full(b1), _full(W2), _full(b2)],
        out_specs=pl.BlockSpec((R, 4), lambda i: (i, 0)),
        out_shape=jax.ShapeDtypeStruct((_N, 4), _f32),
    )(h, W1, b1, W2, b2)


def kernel(x, edge_index, edge_attr, enc_W, enc_b, enc_g, enc_bt, ee_W, ee_b,
           eW1, eb1, eg1, ebt1, eW2, eb2, eg2, ebt2,
           nW1, nb1, ng1, nbt1, nW2, nb2, ng2, nbt2,
           dW1, db1, dW2, db2):
    row3 = edge_index[0].reshape(_NG, _GROUP, _CHUNK)
    col3 = edge_index[1].reshape(_NG, _GROUP, _CHUNK)
    x8 = jnp.pad(x, ((0, 0), (0, 1)))
    W8 = jnp.pad(enc_W, ((0, 1), (0, 0)))

    cnt_raw = _sc_count(col3)
    cnt = cnt_raw.reshape(_NC, _CROWS * 16)[:, :_N].reshape(_NC, _N, 1)

    h = _tc_encode(x8, W8, enc_b, enc_g, enc_bt)
    eW1hb = eW1[:, :2 * _H, :]
    eW1e = eW1[:, 2 * _H:, :]
    e = _tc_edge_encode(edge_attr, ee_W, ee_b)

    L = eW1.shape[0]
    ngc = _NG // _NCH
    ec = ngc * _GE
    rowC = [row3[k * ngc:(k + 1) * ngc] for k in range(_NCH)]
    colC = [col3[k * ngc:(k + 1) * ngc] for k in range(_NCH)]
    eC = [e[k * ec:(k + 1) * ec] for k in range(_NCH)]

    for i in range(L):
        gh = [_sc_gather(h, rowC[k], colC[k]) for k in range(_NCH)]
        aggs = []
        for k in range(_NCH):
            eC[k] = _tc_edge_mlp(gh[k][0], gh[k][1], eC[k], eW1hb[i], eW1e[i],
                                 eb1[i], eg1[i], ebt1[i], eW2[i], eb2[i],
                                 eg2[i], ebt2[i])
            aggs.append(_sc_scatter(eC[k], colC[k]))
        h = _tc_node_mlp(h, aggs, cnt, nW1[i], nb1[i], ng1[i],
                         nbt1[i], nW2[i], nb2[i], ng2[i], nbt2[i])

    return _tc_decode(h, dW1, db1, dW2, db2)


# speedup vs baseline: 1.2123x; 1.0207x over previous
import functools

import jax
import jax.numpy as jnp
from jax import lax
from jax.experimental import pallas as pl
from jax.experimental.pallas import tpu as pltpu
from jax.experimental.pallas import tpu_sc as plsc

_N = 50000
_E = 800000
_H = 64

_NC = 2
_NS = 16
_NW = _NC * _NS

_CHUNK = 128
_GROUP = 5
_GE = _CHUNK * _GROUP
_NR = _E // _CHUNK
_NG = _E // _GE

_HF = _H // _NC
_TROWS = _N // _NS
_CROWS = 3200
_NCH = 5

_f32 = jnp.float32
_bf16 = jnp.bfloat16
_i32 = jnp.int32


def _mesh():
    return plsc.VectorSubcoreMesh(core_axis_name="c", subcore_axis_name="s")


def _sc_gather(h, row2d, col2d):
    ngc = row2d.shape[0]
    ec = ngc * _GE

    @functools.partial(
        pl.kernel,
        out_type=(
            jax.ShapeDtypeStruct((ec, _H), _f32),
            jax.ShapeDtypeStruct((ec, _H), _f32),
        ),
        mesh=_mesh(),
        compiler_params=pltpu.CompilerParams(use_tc_tiling_on_sc=False, needs_layout_passes=False),
        scratch_types=[
            pltpu.VMEM((_GROUP, _CHUNK), _i32),
            pltpu.VMEM((_GROUP, _CHUNK), _i32),
            pltpu.VMEM((_GE, _H), _f32),
            pltpu.VMEM((_GE, _H), _f32),
            pltpu.SemaphoreType.DMA,
            pltpu.SemaphoreType.DMA,
            pltpu.SemaphoreType.DMA,
        ],
    )
    def k(h_hbm, row_hbm, col_hbm, outr, outc, idxA, idxB, bufA, bufB,
          semA, semB, semS):
        c = lax.axis_index("c")
        s = lax.axis_index("s")
        base, rem = divmod(ngc, _NS)
        npairs = base // 2
        ng = base + jnp.where(s < rem, 1, 0)

        def stream(arr_hbm, out_hbm):
            def pair(jj, carry):
                ga = s + (2 * jj) * _NS
                gb = s + (2 * jj + 1) * _NS
                pltpu.sync_copy(arr_hbm.at[ga], idxA)
                dA = [pltpu.async_copy(
                    h_hbm.at[idxA.at[t]], bufA.at[pl.ds(t * _CHUNK, _CHUNK)],
                    semA) for t in range(_GROUP)]
                pltpu.sync_copy(arr_hbm.at[gb], idxB)
                dB = [pltpu.async_copy(
                    h_hbm.at[idxB.at[t]], bufB.at[pl.ds(t * _CHUNK, _CHUNK)],
                    semB) for t in range(_GROUP)]
                for d in dA:
                    d.wait()
                dS = pltpu.async_copy(bufA, out_hbm.at[pl.ds(ga * _GE, _GE)], semS)
                for d in dB:
                    d.wait()
                pltpu.sync_copy(bufB, out_hbm.at[pl.ds(gb * _GE, _GE)])
                dS.wait()
                return carry

            lax.fori_loop(0, npairs, pair, 0)

            def tail(j, carry):
                g = s + j * _NS
                pltpu.sync_copy(arr_hbm.at[g], idxA)
                ds = [pltpu.async_copy(
                    h_hbm.at[idxA.at[t]], bufA.at[pl.ds(t * _CHUNK, _CHUNK)],
                    semA) for t in range(_GROUP)]
                for d in ds:
                    d.wait()
                pltpu.sync_copy(bufA, out_hbm.at[pl.ds(g * _GE, _GE)])
                return carry

            lax.fori_loop(2 * npairs, ng, tail, 0)

        @pl.when(c == 0)
        def _():
            stream(row_hbm, outr)

        @pl.when(c == 1)
        def _():
            stream(col_hbm, outc)

    return k(h, row2d, col2d)


def _sc_scatter(e, col2d):
    QF = 16
    ngc = col2d.shape[0]

    @functools.partial(
        pl.kernel,
        out_type=jax.ShapeDtypeStruct((_N, _H), _f32),
        mesh=_mesh(),
        compiler_params=pltpu.CompilerParams(use_tc_tiling_on_sc=False, needs_layout_passes=False),
        scratch_types=[
            pltpu.VMEM((_GROUP, _CHUNK), _i32),
            pltpu.VMEM((_GROUP, _CHUNK), _i32),
            pltpu.VMEM((_GE, QF), _f32),
            pltpu.VMEM((_GE, QF), _f32),
            pltpu.VMEM((1000, QF), _f32),
            pltpu.VMEM((1000, QF), _f32),
            pltpu.VMEM_SHARED((_N, QF), _f32),
            pltpu.SemaphoreType.DMA,
            pltpu.SemaphoreType.DMA,
            pltpu.SemaphoreType.DMA,
        ],
    )
    def k(e_hbm, col_hbm, out_hbm, idxA, idxB, ebA, ebB, bounce, zbuf, acc,
          semA, semB, semU):
        c = lax.axis_index("c")
        s = lax.axis_index("s")
        z = jnp.zeros((16,), _f32)

        def zb(i, carry):
            zbuf[i, :] = z
            return carry

        lax.fori_loop(0, 1000, zb, 0)

        nz = (_N // 1000 - s + _NS - 1) // _NS

        def zc(j, carry):
            pltpu.sync_copy(zbuf, acc.at[pl.ds((s + j * _NS) * 1000, 1000)])
            return carry

        lax.fori_loop(0, nz, zc, 0)

        base, rem = divmod(ngc, _NS)
        npairs = base // 2
        ng = base + jnp.where(s < rem, 1, 0)

        for p in range(2):
            f0 = (c + 2 * p) * QF
            plsc.subcore_barrier()

            def prefA(g):
                pltpu.async_copy(col_hbm.at[g], idxA, semA)
                pltpu.async_copy(
                    e_hbm.at[pl.ds(g * _GE, _GE), pl.ds(f0, QF)], ebA, semA)

            def waitA(g):
                pltpu.make_async_copy(col_hbm.at[g], idxA, semA).wait()
                pltpu.make_async_copy(
                    e_hbm.at[pl.ds(g * _GE, _GE), pl.ds(f0, QF)], ebA,
                    semA).wait()

            prefA(s)

            def pair(jj, carry):
                ga = s + (2 * jj) * _NS
                gb = ga + _NS
                waitA(ga)
                pltpu.async_copy(col_hbm.at[gb], idxB, semB)
                pltpu.async_copy(
                    e_hbm.at[pl.ds(gb * _GE, _GE), pl.ds(f0, QF)], ebB, semB)
                dU = [pltpu.async_copy(
                    ebA.at[pl.ds(t * _CHUNK, _CHUNK)], acc.at[idxA.at[t]],
                    semU, add=True) for t in range(_GROUP)]
                for d in dU:
                    d.wait()

                @pl.when(jj < npairs - 1)
                def _():
                    prefA(ga + 2 * _NS)

                pltpu.make_async_copy(col_hbm.at[gb], idxB, semB).wait()
                pltpu.make_async_copy(
                    e_hbm.at[pl.ds(gb * _GE, _GE), pl.ds(f0, QF)], ebB,
                    semB).wait()
                dV = [pltpu.async_copy(
                    ebB.at[pl.ds(t * _CHUNK, _CHUNK)], acc.at[idxB.at[t]],
                    semU, add=True) for t in range(_GROUP)]
                for d in dV:
                    d.wait()
                return carry

            lax.fori_loop(0, npairs, pair, 0)

            def tailS(j, carry):
                ga = s + j * _NS
                pltpu.sync_copy(col_hbm.at[ga], idxA)
                pltpu.sync_copy(
                    e_hbm.at[pl.ds(ga * _GE, _GE), pl.ds(f0, QF)], ebA)
                ds = [pltpu.async_copy(
                    ebA.at[pl.ds(t * _CHUNK, _CHUNK)], acc.at[idxA.at[t]],
                    semU, add=True) for t in range(_GROUP)]
                for d in ds:
                    d.wait()
                return carry

            lax.fori_loop(2 * npairs, ng, tailS, 0)

            plsc.subcore_barrier()

            def co(j, carry):
                r0 = (s + j * _NS) * 1000
                pltpu.sync_copy(acc.at[pl.ds(r0, 1000)], bounce)
                if p == 0:
                    pltpu.sync_copy(zbuf, acc.at[pl.ds(r0, 1000)])
                pltpu.sync_copy(bounce, out_hbm.at[pl.ds(r0, 1000), pl.ds(f0, QF)])
                return carry

            lax.fori_loop(0, nz, co, 0)

    return k(e, col2d)


def _sc_count(col2d):
    @functools.partial(
        pl.kernel,
        out_type=jax.ShapeDtypeStruct((_NC, _CROWS, 16), _f32),
        mesh=_mesh(),
        compiler_params=pltpu.CompilerParams(use_tc_tiling_on_sc=False, needs_layout_passes=False),
        scratch_types=[
            pltpu.VMEM((_CROWS, 16), _f32),
            pltpu.VMEM((_GROUP, _CHUNK), _i32),
            pltpu.VMEM((25, 128), _i32),
            pltpu.VMEM((200, 16), _f32),
            pltpu.VMEM_SHARED((_CROWS, 16), _f32),
        ],
    )
    def k(col_hbm, out_hbm, local, idxs, iotaref, bounce, acc):
        c = lax.axis_index("c")
        s = lax.axis_index("s")
        w = s * _NC + c
        z = jnp.zeros((16,), _f32)
        ones = jnp.ones((16,), _f32)
        ar = jnp.arange(16, dtype=_i32)

        def z1(i, carry):
            local[i, :] = z
            return carry

        lax.fori_loop(0, _CROWS, z1, 0)

        def z2(i, carry):
            bounce[i, :] = z
            return carry

        lax.fori_loop(0, 200, z2, 0)
        pltpu.sync_copy(bounce, acc.at[pl.ds(s * 200, 200)])

        def bi(j, carry):
            for t in range(8):
                iotaref[j, pl.ds(t * 16, 16)] = ar + (j * 128 + t * 16)
            return carry

        lax.fori_loop(0, 25, bi, 0)
        plsc.subcore_barrier()

        base, rem = divmod(_NG, _NW)
        ngr = base + jnp.where(w < rem, 1, 0)

        def body(j, carry):
            g = w + j * _NW
            pltpu.sync_copy(col_hbm.at[g], idxs)
            for t in range(_GROUP):
                for q in range(_CHUNK // 16):
                    iv = idxs[t, pl.ds(q * 16, 16)]
                    rr = lax.shift_right_logical(iv, 4)
                    cc = lax.bitwise_and(iv, 15)
                    plsc.addupdate_scatter(local, [rr, cc], ones)
            return carry

        lax.fori_loop(0, ngr, body, 0)

        def mg(j, carry):
            pltpu.sync_copy(
                local.at[pl.ds(j * 128, 128)], acc.at[iotaref.at[j]], add=True)
            return carry

        lax.fori_loop(0, 25, mg, 0)
        plsc.subcore_barrier()

        pltpu.sync_copy(acc.at[pl.ds(s * 200, 200)], bounce)
        pltpu.sync_copy(bounce, out_hbm.at[c, pl.ds(s * 200, 200)])

    return k(col2d)


def _gelu(x):
    return x * 0.5 * (1.0 + lax.erf(x * 0.7071067811865476))


def _lnorm(x, g, b):
    n = x.shape[-1]
    sx = jnp.sum(x, axis=-1, keepdims=True)
    sxx = jnp.sum(x * x, axis=-1, keepdims=True)
    m = sx * (1.0 / n)
    v = sxx * (1.0 / n) - m * m
    r = lax.rsqrt(v + 1e-5)
    return (x - m) * (r * g) + b


def _full(a):
    return pl.BlockSpec(a.shape, lambda i: (0,) * a.ndim)


def _tc_encode(x8, W8, b, g, bt):
    R = 1000

    def body(x_r, W_r, b_r, g_r, bt_r, out_r):
        t = jnp.dot(x_r[...], W_r[...], preferred_element_type=_f32) + b_r[...]
        out_r[...] = _gelu(_lnorm(t, g_r[...], bt_r[...]))

    return pl.pallas_call(
        body,
        grid=(_N // R,),
        in_specs=[pl.BlockSpec((R, 8), lambda i: (i, 0)),
                  _full(W8), _full(b), _full(g), _full(bt)],
        out_specs=pl.BlockSpec((R, _H), lambda i: (i, 0)),
        out_shape=jax.ShapeDtypeStruct((_N, _H), _f32),
    )(x8, W8, b, g, bt)


def _tc_edge_encode(ea, W, b):
    R = 1600

    def body(a_r, W_r, b_r, out_r):
        out_r[...] = jnp.dot(a_r[...], W_r[...], preferred_element_type=_f32) + b_r[...]

    return pl.pallas_call(
        body,
        grid=(_E // R,),
        in_specs=[pl.BlockSpec((R, 8), lambda i: (i, 0)), _full(W), _full(b)],
        out_specs=pl.BlockSpec((R, _H), lambda i: (i, 0)),
        out_shape=jax.ShapeDtypeStruct((_E, _H), _f32),
    )(ea, W, b)


def _tc_edge_mlp(ghr, ghc, e, W1hb, W1e, b1, g1, bt1, W2, b2, g2, bt2):
    R = 6400

    def body(ghr_r, ghc_r, e_r, W1hb_r, W1e_r, b1_r, g1_r, bt1_r, W2_r, b2_r,
             g2_r, bt2_r, out_r):
        gin = jnp.concatenate([ghr_r[...], ghc_r[...]], axis=1)
        t = (jnp.dot(gin, W1hb_r[...], preferred_element_type=_f32)
             + jnp.dot(e_r[...], W1e_r[...], preferred_element_type=_f32)
             + b1_r[...])
        t = _gelu(_lnorm(t, g1_r[...], bt1_r[...]))
        t = jnp.dot(t, W2_r[...], preferred_element_type=_f32) + b2_r[...]
        t = _lnorm(t, g2_r[...], bt2_r[...])
        out_r[...] = e_r[...] + t

    ec = e.shape[0]
    blk = pl.BlockSpec((R, _H), lambda i: (i, 0))
    return pl.pallas_call(
        body,
        grid=(ec // R,),
        in_specs=[blk, blk, blk, _full(W1hb), _full(W1e), _full(b1),
                  _full(g1), _full(bt1),
                  _full(W2), _full(b2), _full(g2), _full(bt2)],
        out_specs=blk,
        out_shape=jax.ShapeDtypeStruct((ec, _H), _f32),
    )(ghr, ghc, e, W1hb, W1e, b1, g1, bt1, W2, b2, g2, bt2)


def _tc_node_mlp(h, aggs, cnt, W1, b1, g1, bt1, W2, b2, g2, bt2):
    R = 2000
    nagg = len(aggs)

    def body(h_r, *rest):
        agg_rs = rest[:nagg]
        (cnt_r, W1_r, b1_r, g1_r, bt1_r, W2_r, b2_r, g2_r, bt2_r,
         out_r) = rest[nagg:]
        csum = cnt_r[0] + cnt_r[1]
        inv = 1.0 / jnp.maximum(csum, 1.0)
        agg = agg_rs[0][...]
        for a in agg_rs[1:]:
            agg = agg + a[...]
        nin = jnp.concatenate([h_r[...], agg * inv], axis=1)
        u = jnp.dot(nin, W1_r[...], preferred_element_type=_f32) + b1_r[...]
        u = _gelu(_lnorm(u, g1_r[...], bt1_r[...]))
        u = jnp.dot(u, W2_r[...], preferred_element_type=_f32) + b2_r[...]
        u = _lnorm(u, g2_r[...], bt2_r[...])
        out_r[...] = h_r[...] + u

    blk = pl.BlockSpec((R, _H), lambda i: (i, 0))
    return pl.pallas_call(
        body,
        grid=(_N // R,),
        in_specs=[blk] * (1 + nagg)
        + [pl.BlockSpec((2, R, 1), lambda i: (0, i, 0)),
           _full(W1), _full(b1), _full(g1), _full(bt1),
           _full(W2), _full(b2), _full(g2), _full(bt2)],
        out_specs=blk,
        out_shape=jax.ShapeDtypeStruct((_N, _H), _f32),
    )(h, *aggs, cnt, W1, b1, g1, bt1, W2, b2, g2, bt2)


def _tc_decode(h, W1, b1, W2, b2):
    R = 1000

    def body(h_r, W1_r, b1_r, W2_r, b2_r, out_r):
        t = _gelu(jnp.dot(h_r[...], W1_r[...], preferred_element_type=_f32)
                  + b1_r[...])
        out_r[...] = jnp.dot(t, W2_r[...], preferred_element_type=_f32) + b2_r[...]

    return pl.pallas_call(
        body,
        grid=(_N // R,),
        in_specs=[pl.BlockSpec((R, _H), lambda i: (i, 0)),
                  _full(W1), _full(b1), _full(W2), _full(b2)],
        out_specs=pl.BlockSpec((R, 4), lambda i: (i, 0)),
        out_shape=jax.ShapeDtypeStruct((_N, 4), _f32),
    )(h, W1, b1, W2, b2)


def kernel(x, edge_index, edge_attr, enc_W, enc_b, enc_g, enc_bt, ee_W, ee_b,
           eW1, eb1, eg1, ebt1, eW2, eb2, eg2, ebt2,
           nW1, nb1, ng1, nbt1, nW2, nb2, ng2, nbt2,
           dW1, db1, dW2, db2):
    row3 = edge_index[0].reshape(_NG, _GROUP, _CHUNK)
    col3 = edge_index[1].reshape(_NG, _GROUP, _CHUNK)
    x8 = jnp.pad(x, ((0, 0), (0, 1)))
    W8 = jnp.pad(enc_W, ((0, 1), (0, 0)))

    cnt_raw = _sc_count(col3)
    cnt = cnt_raw.reshape(_NC, _CROWS * 16)[:, :_N].reshape(_NC, _N, 1)

    h = _tc_encode(x8, W8, enc_b, enc_g, enc_bt)
    eW1hb = eW1[:, :2 * _H, :]
    eW1e = eW1[:, 2 * _H:, :]
    e = _tc_edge_encode(edge_attr, ee_W, ee_b)

    L = eW1.shape[0]
    ngc = _NG // _NCH
    ec = ngc * _GE
    rowC = [row3[k * ngc:(k + 1) * ngc] for k in range(_NCH)]
    colC = [col3[k * ngc:(k + 1) * ngc] for k in range(_NCH)]
    eC = [e[k * ec:(k + 1) * ec] for k in range(_NCH)]

    for i in range(L):
        gh = [_sc_gather(h, rowC[k], colC[k]) for k in range(_NCH)]
        aggs = []
        for k in range(_NCH):
            eC[k] = _tc_edge_mlp(gh[k][0], gh[k][1], eC[k], eW1hb[i], eW1e[i],
                                 eb1[i], eg1[i], ebt1[i], eW2[i], eb2[i],
                                 eg2[i], ebt2[i])
            aggs.append(_sc_scatter(eC[k], colC[k]))
        h = _tc_node_mlp(h, aggs, cnt, nW1[i], nb1[i], ng1[i],
                         nbt1[i], nW2[i], nb2[i], ng2[i], nbt2[i])

    return _tc_decode(h, dW1, db1, dW2, db2)
